# Initial kernel scaffold; baseline (speedup 1.0000x reference)
import jax
import jax.numpy as jnp
from jax.experimental import pallas as pl


def kernel(x, edge_index, time, node_time, batch_size, params):
    raise NotImplementedError("write your pallas kernel here")


# speedup vs baseline: 1.0091x; 1.0091x over previous
import functools

import jax
import jax.numpy as jnp
from jax.experimental import pallas as pl

N = 10000
E = 160000
IN = 128
HID = 256
HEADS = 4
C = HID // HEADS
TD = 64
NL = 2
BS = 4096


def _matmul_relu_kernel(x_ref, w_ref, b_ref, o_ref):
    o_ref[...] = jax.nn.relu(
        jnp.dot(x_ref[...], w_ref[...], preferred_element_type=jnp.float32)
        + b_ref[...]
    )


def _input_proj(x, w, b):
    blk = 2000
    return pl.pallas_call(
        _matmul_relu_kernel,
        grid=(N // blk,),
        in_specs=[
            pl.BlockSpec((blk, IN), lambda i: (i, 0)),
            pl.BlockSpec((IN, HID), lambda i: (0, 0)),
            pl.BlockSpec((1, HID), lambda i: (0, 0)),
        ],
        out_specs=pl.BlockSpec((blk, HID), lambda i: (i, 0)),
        out_shape=jax.ShapeDtypeStruct((N, HID), jnp.float32),
    )(x, w, b.reshape(1, HID))


def _bn_eval(h, g, b):
    eps = 1e-5
    return (h / jnp.sqrt(1.0 + eps)) * g + b


def kernel(x, edge_index, time, node_time, batch_size, params):
    src = edge_index[0]
    dst = edge_index[1]
    rel_t = node_time[dst] - time
    e_enc = jnp.cos(rel_t[:, None] * params['basis_freq'][None, :] + params['phase'][None, :])
    h = _input_proj(x, params['W_in'], params['b_in'])
    scale = 1.0 / jnp.sqrt(jnp.asarray(C, jnp.float32))
    for l in range(NL):
        q = (h @ params['Wq'][l] + params['bq'][l])[dst].reshape(E, HEADS, C)
        k = (h @ params['Wk'][l] + params['bk'][l])[src].reshape(E, HEADS, C)
        v = (h @ params['Wv'][l] + params['bv'][l])[src].reshape(E, HEADS, C)
        e = (e_enc @ params['We'][l]).reshape(E, HEADS, C)
        k = k + e
        alpha = (q * k).sum(-1) * scale
        amax = jax.ops.segment_max(alpha, dst, num_segments=N)
        alpha = jnp.exp(alpha - amax[dst])
        denom = jax.ops.segment_sum(alpha, dst, num_segments=N)
        alpha = alpha / (denom[dst] + 1e-16)
        msg = (v + e) * alpha[..., None]
        out = jax.ops.segment_sum(msg.reshape(E, HID), dst, num_segments=N)
        out = out + h @ params['Wskip'][l] + params['bskip'][l]
        h = _bn_eval(jax.nn.relu(out), params['gamma'][l], params['beta'][l])
    z = jax.lax.dynamic_slice_in_dim(h, batch_size - BS, BS, axis=0)
    z = jax.nn.relu(_bn_eval(z @ params['W1'] + params['b1'], params['g1'], params['be1']))
    z = jax.nn.relu(_bn_eval(z @ params['W2'] + params['b2'], params['g2'], params['be2']))
    z = z @ params['W3'] + params['b3']
    return z[:, 0]


# speedup vs baseline: 1.3140x; 1.3022x over previous
import functools

import jax
import jax.numpy as jnp
from jax import lax
from jax.experimental import pallas as pl
from jax.experimental.pallas import tpu as pltpu
from jax.experimental.pallas import tpu_sc as plsc

N = 10000
E = 160000
IN = 128
HID = 256
HEADS = 4
C = HID // HEADS
TD = 64
NL = 2
BS = 4096

NC = 2
NS = 16
NW = NC * NS
EP = 163840
PER_W = EP // NW
GB = 128
PER_T = EP // NS
SB = 128
QW0 = 384
QW1 = 256
SW = 128
QUART = 2560
ACC_ROWS = 2688
DRAIN = ACC_ROWS // NS

_EPS_BN = 1e-5
_BN_SCALE = 1.0 / (1.0 + _EPS_BN) ** 0.5


def _bn(h, g, b):
    return h * (g * _BN_SCALE) + b


def _inproj_body(x_ref, w_ref, b_ref, o_ref):
    o_ref[...] = jax.nn.relu(
        jnp.dot(x_ref[...], w_ref[...], preferred_element_type=jnp.float32)
        + b_ref[...])


def _input_proj(x, w, b):
    blk = 2000
    return pl.pallas_call(
        _inproj_body,
        grid=(N // blk,),
        in_specs=[
            pl.BlockSpec((blk, IN), lambda i: (i, 0)),
            pl.BlockSpec((IN, HID), lambda i: (0, 0)),
            pl.BlockSpec((1, HID), lambda i: (0, 0)),
        ],
        out_specs=pl.BlockSpec((blk, HID), lambda i: (i, 0)),
        out_shape=jax.ShapeDtypeStruct((N, HID), jnp.float32),
    )(x, w, b.reshape(1, HID))


def _make_proj_body(qw):
    def body(h_ref, nt_ref, wq_ref, bq_ref, wk_ref, bk_ref, wv_ref, bv_ref,
             qt_ref, kvt_ref):
        h = h_ref[...]
        q = jnp.dot(h, wq_ref[...], preferred_element_type=jnp.float32) + bq_ref[...]
        k = jnp.dot(h, wk_ref[...], preferred_element_type=jnp.float32) + bk_ref[...]
        v = jnp.dot(h, wv_ref[...], preferred_element_type=jnp.float32) + bv_ref[...]
        if qw > HID:
            pad = jnp.zeros((h.shape[0], qw - HID - 1), jnp.float32)
            qt_ref[...] = jnp.concatenate([q, nt_ref[...], pad], axis=1)
        else:
            qt_ref[...] = q
        kvt_ref[...] = jnp.concatenate([k, v], axis=1)
    return body


def _projections(h, nt, wq, bq, wk, bk, wv, bv, qw):
    blk = 1000
    w_spec = pl.BlockSpec((HID, HID), lambda i: (0, 0))
    b_spec = pl.BlockSpec((1, HID), lambda i: (0, 0))
    return pl.pallas_call(
        _make_proj_body(qw),
        grid=(N // blk,),
        in_specs=[
            pl.BlockSpec((blk, HID), lambda i: (i, 0)),
            pl.BlockSpec((blk, 1), lambda i: (i, 0)),
            w_spec, b_spec, w_spec, b_spec, w_spec, b_spec,
        ],
        out_specs=[
            pl.BlockSpec((blk, qw), lambda i: (i, 0)),
            pl.BlockSpec((blk, 2 * HID), lambda i: (i, 0)),
        ],
        out_shape=[
            jax.ShapeDtypeStruct((N, qw), jnp.float32),
            jax.ShapeDtypeStruct((N, 2 * HID), jnp.float32),
        ],
    )(h, nt, wq, bq.reshape(1, HID), wk, bk.reshape(1, HID),
      wv, bv.reshape(1, HID))


_BE = 2048


def _enc(rel_t, bf_ref, ph_ref):
    return jnp.cos(rel_t * bf_ref[...] + ph_ref[...])


def _make_alpha_body(qw):
    def body(qd_ref, k_ref, rt_ref, t_ref, bf_ref, ph_ref, we_ref,
             ax_ref, bm_ref):
        qd = qd_ref[...]
        if qw > HID:
            rel_t = qd[:, HID:HID + 1] - t_ref[...]
        else:
            rel_t = rt_ref[...]
        em = jnp.dot(_enc(rel_t, bf_ref, ph_ref), we_ref[...],
                     preferred_element_type=jnp.float32)
        kk = k_ref[...] + em
        prod = (qd[:, :HID] * kk).reshape(_BE, HEADS, C)
        scale = 1.0 / (C ** 0.5)
        alpha = prod.sum(axis=-1) * scale
        pad = jnp.zeros((_BE, 3), jnp.float32)
        ax_ref[...] = jnp.concatenate([alpha, rel_t, pad], axis=1)
        bm_ref[...] = jnp.max(alpha, axis=0).reshape(1, 1, HEADS)
    return body


def _alpha_pass(qd, kvs, rt_col, t_col, bf, ph, we, qw):
    grid = EP // _BE
    return pl.pallas_call(
        _make_alpha_body(qw),
        grid=(grid,),
        in_specs=[
            pl.BlockSpec((_BE, qw), lambda i: (i, 0)),
            pl.BlockSpec((_BE, HID), lambda i: (i, 0)),
            pl.BlockSpec((_BE, 1), lambda i: (i, 0)),
            pl.BlockSpec((_BE, 1), lambda i: (i, 0)),
            pl.BlockSpec((1, TD), lambda i: (0, 0)),
            pl.BlockSpec((1, TD), lambda i: (0, 0)),
            pl.BlockSpec((TD, HID), lambda i: (0, 0)),
        ],
        out_specs=[
            pl.BlockSpec((_BE, 8), lambda i: (i, 0)),
            pl.BlockSpec((1, 1, HEADS), lambda i: (i, 0, 0)),
        ],
        out_shape=[
            jax.ShapeDtypeStruct((EP, 8), jnp.float32),
            jax.ShapeDtypeStruct((grid, 1, HEADS), jnp.float32),
        ],
    )(qd, kvs, rt_col, t_col, bf.reshape(1, TD), ph.reshape(1, TD), we)


def _msg_body(ax_ref, gm_ref, v_ref, bf_ref, ph_ref, we_ref,
              lo_ref, hi_ref, wp_ref):
    ax = ax_ref[...]
    rel_t = ax[:, HEADS:HEADS + 1]
    em = jnp.dot(_enc(rel_t, bf_ref, ph_ref), we_ref[...],
                 preferred_element_type=jnp.float32)
    w = jnp.exp(ax[:, :HEADS] - gm_ref[...])
    wb = jnp.broadcast_to(w.reshape(_BE, HEADS, 1), (_BE, HEADS, C))
    msg = (v_ref[...] + em) * wb.reshape(_BE, HID)
    lo_ref[...] = msg[:, :SW]
    hi_ref[...] = msg[:, SW:]
    wp_ref[...] = jnp.concatenate(
        [w, jnp.zeros((_BE, SW - HEADS), jnp.float32)], axis=1)


def _msg_pass(ax, gmax, kvs, bf, ph, we):
    grid = EP // _BE
    return pl.pallas_call(
        _msg_body,
        grid=(grid,),
        in_specs=[
            pl.BlockSpec((_BE, 8), lambda i: (i, 0)),
            pl.BlockSpec((1, HEADS), lambda i: (0, 0)),
            pl.BlockSpec((_BE, HID), lambda i: (i, 1)),
            pl.BlockSpec((1, TD), lambda i: (0, 0)),
            pl.BlockSpec((1, TD), lambda i: (0, 0)),
            pl.BlockSpec((TD, HID), lambda i: (0, 0)),
        ],
        out_specs=[
            pl.BlockSpec((_BE, SW), lambda i: (i, 0)),
            pl.BlockSpec((_BE, SW), lambda i: (i, 0)),
            pl.BlockSpec((_BE, SW), lambda i: (i, 0)),
        ],
        out_shape=[
            jax.ShapeDtypeStruct((EP, SW), jnp.float32),
            jax.ShapeDtypeStruct((EP, SW), jnp.float32),
            jax.ShapeDtypeStruct((EP, SW), jnp.float32),
        ],
    )(ax, gmax, kvs, bf.reshape(1, TD), ph.reshape(1, TD), we)


def _hupd_body(lo_ref, hi_ref, wp_ref, h_ref, ws_ref, bs_ref, g_ref, be_ref,
               o_ref):
    lo = lo_ref[0]
    hi = hi_ref[0]
    blk = lo.shape[0]
    msg = jnp.concatenate([lo, hi], axis=1)
    den = wp_ref[0][:, :HEADS]
    den_b = jnp.broadcast_to(den.reshape(blk, HEADS, 1), (blk, HEADS, C))
    den_b = den_b.reshape(blk, HID)
    out = msg / jnp.maximum(den_b, 1e-30)
    out = out + jnp.dot(h_ref[...], ws_ref[...],
                        preferred_element_type=jnp.float32) + bs_ref[...]
    o_ref[...] = _bn(jax.nn.relu(out), g_ref[...], be_ref[...])


def _h_update_half(lo3, hi3, wp3, h, ws, bs, g, be, p):
    blk = 512
    k = QUART // blk
    acc_spec = pl.BlockSpec((1, blk, SW), lambda c, i: (c, i, 0))
    return pl.pallas_call(
        _hupd_body,
        grid=(NC, k),
        in_specs=[
            acc_spec, acc_spec, acc_spec,
            pl.BlockSpec((blk, HID), lambda c, i: (p * 2 * k + c * k + i, 0)),
            pl.BlockSpec((HID, HID), lambda c, i: (0, 0)),
            pl.BlockSpec((1, HID), lambda c, i: (0, 0)),
            pl.BlockSpec((1, HID), lambda c, i: (0, 0)),
            pl.BlockSpec((1, HID), lambda c, i: (0, 0)),
        ],
        out_specs=pl.BlockSpec((blk, HID), lambda c, i: (c * k + i, 0)),
        out_shape=jax.ShapeDtypeStruct((2 * QUART, HID), jnp.float32),
    )(lo3, hi3, wp3, h, ws, bs.reshape(1, HID), g.reshape(1, HID),
      be.reshape(1, HID))


def _head_body(h_ref, w1_ref, b1_ref, g1_ref, e1_ref, w2_ref, b2_ref,
               g2_ref, e2_ref, w3_ref, b3_ref, o_ref):
    z = jnp.dot(h_ref[...], w1_ref[...], preferred_element_type=jnp.float32)
    z = jax.nn.relu(_bn(z + b1_ref[...], g1_ref[...], e1_ref[...]))
    z = jnp.dot(z, w2_ref[...], preferred_element_type=jnp.float32)
    z = jax.nn.relu(_bn(z + b2_ref[...], g2_ref[...], e2_ref[...]))
    o_ref[...] = jnp.dot(z, w3_ref[...],
                         preferred_element_type=jnp.float32) + b3_ref[...]


def _head(h, p):
    blk = 512
    h2 = HID // 2
    return pl.pallas_call(
        _head_body,
        grid=(BS // blk,),
        in_specs=[
            pl.BlockSpec((blk, HID), lambda i: (i, 0)),
            pl.BlockSpec((HID, HID), lambda i: (0, 0)),
            pl.BlockSpec((1, HID), lambda i: (0, 0)),
            pl.BlockSpec((1, HID), lambda i: (0, 0)),
            pl.BlockSpec((1, HID), lambda i: (0, 0)),
            pl.BlockSpec((HID, h2), lambda i: (0, 0)),
            pl.BlockSpec((1, h2), lambda i: (0, 0)),
            pl.BlockSpec((1, h2), lambda i: (0, 0)),
            pl.BlockSpec((1, h2), lambda i: (0, 0)),
            pl.BlockSpec((h2, 1), lambda i: (0, 0)),
            pl.BlockSpec((1, 1), lambda i: (0, 0)),
        ],
        out_specs=pl.BlockSpec((blk, 1), lambda i: (i, 0)),
        out_shape=jax.ShapeDtypeStruct((BS, 1), jnp.float32),
    )(h, p['W1'], p['b1'].reshape(1, HID), p['g1'].reshape(1, HID),
      p['be1'].reshape(1, HID), p['W2'], p['b2'].reshape(1, h2),
      p['g2'].reshape(1, h2), p['be2'].reshape(1, h2),
      p['W3'], p['b3'].reshape(1, 1))


_MESH = plsc.VectorSubcoreMesh(core_axis_name="c", subcore_axis_name="s")
_SC_PARAMS = pltpu.CompilerParams(needs_layout_passes=False)


def _make_sc_gather(qw):
    @functools.partial(
        pl.kernel,
        out_type=[
            jax.ShapeDtypeStruct((EP, qw), jnp.float32),
            jax.ShapeDtypeStruct((EP, 2 * HID), jnp.float32),
        ],
        mesh=_MESH,
        scratch_types=[
            pltpu.VMEM((GB,), jnp.int32),
            pltpu.VMEM((GB,), jnp.int32),
            pltpu.VMEM((GB, qw), jnp.float32),
            pltpu.VMEM((GB, 2 * HID), jnp.float32),
            pltpu.SemaphoreType.DMA,
            pltpu.SemaphoreType.DMA,
        ],
        compiler_params=_SC_PARAMS,
    )
    def sc_gather(qt_hbm, kvt_hbm, dg_hbm, sg_hbm,
                  qd_out, kvs_out, di_v, si_v, qrows, kvrows, sem1, sem2):
        wid = lax.axis_index("s") * NC + lax.axis_index("c")
        base = wid * PER_W

        def chunk(j, carry):
            off = base + j * GB
            pltpu.sync_copy(dg_hbm.at[pl.ds(off, GB)], di_v)
            pltpu.sync_copy(sg_hbm.at[pl.ds(off, GB)], si_v)
            cp1 = pltpu.async_copy(qt_hbm.at[di_v], qrows, sem1)
            cp2 = pltpu.async_copy(kvt_hbm.at[si_v], kvrows, sem2)
            cp1.wait()
            cp2.wait()
            pltpu.sync_copy(qrows, qd_out.at[pl.ds(off, GB)])
            pltpu.sync_copy(kvrows, kvs_out.at[pl.ds(off, GB)])
            return carry

        lax.fori_loop(0, PER_W // GB, chunk, 0)

    return sc_gather


_sc_gather0 = _make_sc_gather(QW0)
_sc_gather1 = _make_sc_gather(QW1)


def _make_sc_scatter(p):
    acc_t = jax.ShapeDtypeStruct((NC, ACC_ROWS, SW), jnp.float32)

    @functools.partial(
        pl.kernel,
        out_type=[acc_t, acc_t, acc_t],
        mesh=_MESH,
        scratch_types=[
            pltpu.VMEM((SB,), jnp.int32),
            pltpu.VMEM((SB,), jnp.int32),
            pltpu.VMEM((SB, SW), jnp.float32),
            pltpu.VMEM((SB, SW), jnp.float32),
            pltpu.VMEM((SB, SW), jnp.float32),
            pltpu.VMEM_SHARED((ACC_ROWS, SW), jnp.float32),
            pltpu.VMEM_SHARED((ACC_ROWS, SW), jnp.float32),
            pltpu.VMEM_SHARED((ACC_ROWS, SW), jnp.float32),
        ],
        compiler_params=_SC_PARAMS,
    )
    def sc_scatter(lo_hbm, hi_hbm, wp_hbm, ds_hbm, zrows_hbm,
                   lo_out, hi_out, wp_out,
                   di_v, ai_v, lo_v, hi_v, wp_v, lo_sh, hi_sh, wp_sh):
        cid = lax.axis_index("c")
        sid = lax.axis_index("s")
        nbase = (2 * p + cid) * QUART

        zslice = pl.ds(sid * DRAIN, DRAIN)
        pltpu.sync_copy(zrows_hbm, lo_sh.at[zslice])
        pltpu.sync_copy(zrows_hbm, hi_sh.at[zslice])
        pltpu.sync_copy(zrows_hbm, wp_sh.at[zslice])
        plsc.subcore_barrier()

        def chunk(j, carry):
            off = sid * PER_T + j * SB
            sl_rows = pl.ds(off, SB)
            pltpu.sync_copy(ds_hbm.at[sl_rows], di_v)
            pltpu.sync_copy(lo_hbm.at[sl_rows], lo_v)
            pltpu.sync_copy(hi_hbm.at[sl_rows], hi_v)
            pltpu.sync_copy(wp_hbm.at[sl_rows], wp_v)
            for k in range(SB // 16):
                sl = pl.ds(k * 16, 16)
                rel = di_v[sl] - nbase
                ok = (rel >= 0) & (rel < QUART)
                ai_v[sl] = jnp.where(ok, rel, QUART)
            pltpu.sync_copy(lo_v, lo_sh.at[ai_v], add=True)
            pltpu.sync_copy(hi_v, hi_sh.at[ai_v], add=True)
            pltpu.sync_copy(wp_v, wp_sh.at[ai_v], add=True)
            return carry

        lax.fori_loop(0, PER_T // SB, chunk, 0)
        plsc.subcore_barrier()
        pltpu.sync_copy(lo_sh.at[zslice], lo_out.at[cid, zslice])
        pltpu.sync_copy(hi_sh.at[zslice], hi_out.at[cid, zslice])
        pltpu.sync_copy(wp_sh.at[zslice], wp_out.at[cid, zslice])

    return sc_scatter


_sc_scatter0 = _make_sc_scatter(0)
_sc_scatter1 = _make_sc_scatter(1)


def kernel(x, edge_index, time, node_time, batch_size, params):
    src = edge_index[0]
    dst = edge_index[1]
    pad = EP - E
    dg = jnp.concatenate([dst, jnp.zeros((pad,), jnp.int32)])
    sg = jnp.concatenate([src, jnp.zeros((pad,), jnp.int32)])
    ds_ = jnp.concatenate([dst, jnp.full((pad,), jnp.int32(2 ** 20))])
    t_col = jnp.concatenate([time, jnp.zeros((pad,), jnp.float32)]).reshape(EP, 1)
    nt_col = node_time.reshape(N, 1)
    zrows = jnp.zeros((DRAIN, SW), jnp.float32)

    p = params
    h = _input_proj(x, p['W_in'], p['b_in'])
    rt_col = t_col
    for l in range(NL):
        qw = QW0 if l == 0 else QW1
        qt, kvt = _projections(h, nt_col, p['Wq'][l], p['bq'][l],
                               p['Wk'][l], p['bk'][l], p['Wv'][l], p['bv'][l],
                               qw)
        gather = _sc_gather0 if l == 0 else _sc_gather1
        qd, kvs = gather(qt, kvt, dg, sg)
        ax, bmax = _alpha_pass(qd, kvs, rt_col, t_col,
                               p['basis_freq'], p['phase'], p['We'][l], qw)
        rt_col = ax[:, HEADS:HEADS + 1]
        gmax = jnp.max(bmax[:, 0, :], axis=0, keepdims=True)
        lo, hi, wp = _msg_pass(ax, gmax, kvs, p['basis_freq'], p['phase'],
                               p['We'][l])
        lo_a, hi_a, wp_a = _sc_scatter0(lo, hi, wp, ds_, zrows)
        lo_b, hi_b, wp_b = _sc_scatter1(lo, hi, wp, ds_, zrows)
        h_a = _h_update_half(lo_a, hi_a, wp_a, h, p['Wskip'][l], p['bskip'][l],
                             p['gamma'][l], p['beta'][l], 0)
        h_b = _h_update_half(lo_b, hi_b, wp_b, h, p['Wskip'][l], p['bskip'][l],
                             p['gamma'][l], p['beta'][l], 1)
        h = jnp.concatenate([h_a, h_b[:N - 2 * QUART]], axis=0)
    z = _head(h, p)
    return z[:, 0]


# speedup vs baseline: 1.4796x; 1.1260x over previous
import functools

import jax
import jax.numpy as jnp
from jax import lax
from jax.experimental import pallas as pl
from jax.experimental.pallas import tpu as pltpu
from jax.experimental.pallas import tpu_sc as plsc

N = 10000
E = 160000
IN = 128
HID = 256
HEADS = 4
C = HID // HEADS
TD = 64
NL = 2
BS = 4096

NC = 2
NS = 16
NW = NC * NS
EP = 163840
PER_W = EP // NW
GB = 64
PER_T = EP // NS
SB = 64
QW0 = 384
QW1 = 256
SW = 128
QUART = 2560
ACC_ROWS = 2688
DRAIN = ACC_ROWS // NS

_EPS_BN = 1e-5
_BN_SCALE = 1.0 / (1.0 + _EPS_BN) ** 0.5


def _bn(h, g, b):
    return h * (g * _BN_SCALE) + b


def _inproj_body(x_ref, w_ref, b_ref, o_ref):
    o_ref[...] = jax.nn.relu(
        jnp.dot(x_ref[...], w_ref[...], preferred_element_type=jnp.float32)
        + b_ref[...])


def _input_proj(x, w, b):
    blk = 2000
    return pl.pallas_call(
        _inproj_body,
        grid=(N // blk,),
        in_specs=[
            pl.BlockSpec((blk, IN), lambda i: (i, 0)),
            pl.BlockSpec((IN, HID), lambda i: (0, 0)),
            pl.BlockSpec((1, HID), lambda i: (0, 0)),
        ],
        out_specs=pl.BlockSpec((blk, HID), lambda i: (i, 0)),
        out_shape=jax.ShapeDtypeStruct((N, HID), jnp.float32),
    )(x, w, b.reshape(1, HID))


def _make_proj_body(qw):
    def body(h_ref, nt_ref, wq_ref, bq_ref, wk_ref, bk_ref, wv_ref, bv_ref,
             qt_ref, kvt_ref):
        h = h_ref[...]
        q = jnp.dot(h, wq_ref[...], preferred_element_type=jnp.float32) + bq_ref[...]
        k = jnp.dot(h, wk_ref[...], preferred_element_type=jnp.float32) + bk_ref[...]
        v = jnp.dot(h, wv_ref[...], preferred_element_type=jnp.float32) + bv_ref[...]
        if qw > HID:
            pad = jnp.zeros((h.shape[0], qw - HID - 1), jnp.float32)
            qt_ref[...] = jnp.concatenate([q, nt_ref[...], pad], axis=1)
        else:
            qt_ref[...] = q
        kvt_ref[...] = jnp.concatenate([k, v], axis=1)
    return body


def _projections(h, nt, wq, bq, wk, bk, wv, bv, qw):
    blk = 1000
    w_spec = pl.BlockSpec((HID, HID), lambda i: (0, 0))
    b_spec = pl.BlockSpec((1, HID), lambda i: (0, 0))
    return pl.pallas_call(
        _make_proj_body(qw),
        grid=(N // blk,),
        in_specs=[
            pl.BlockSpec((blk, HID), lambda i: (i, 0)),
            pl.BlockSpec((blk, 1), lambda i: (i, 0)),
            w_spec, b_spec, w_spec, b_spec, w_spec, b_spec,
        ],
        out_specs=[
            pl.BlockSpec((blk, qw), lambda i: (i, 0)),
            pl.BlockSpec((blk, 2 * HID), lambda i: (i, 0)),
        ],
        out_shape=[
            jax.ShapeDtypeStruct((N, qw), jnp.float32),
            jax.ShapeDtypeStruct((N, 2 * HID), jnp.float32),
        ],
    )(h, nt, wq, bq.reshape(1, HID), wk, bk.reshape(1, HID),
      wv, bv.reshape(1, HID))


_BE = 2048


def _enc(rel_t, bf_ref, ph_ref):
    return jnp.cos(rel_t * bf_ref[...] + ph_ref[...])


def _make_alpha_body(qw):
    def body(qd_ref, k_ref, rt_ref, t_ref, bf_ref, ph_ref, we_ref,
             ax_ref, bm_ref):
        qd = qd_ref[...]
        if qw > HID:
            rel_t = qd[:, HID:HID + 1] - t_ref[...]
        else:
            rel_t = rt_ref[...]
        em = jnp.dot(_enc(rel_t, bf_ref, ph_ref), we_ref[...],
                     preferred_element_type=jnp.float32)
        kk = k_ref[...] + em
        prod = (qd[:, :HID] * kk).reshape(_BE, HEADS, C)
        scale = 1.0 / (C ** 0.5)
        alpha = prod.sum(axis=-1) * scale
        pad = jnp.zeros((_BE, 3), jnp.float32)
        ax_ref[...] = jnp.concatenate([alpha, rel_t, pad], axis=1)
        bm_ref[...] = jnp.max(alpha, axis=0).reshape(1, 1, HEADS)
    return body


def _alpha_pass(qd, kvs, rt_col, t_col, bf, ph, we, qw):
    grid = EP // _BE
    return pl.pallas_call(
        _make_alpha_body(qw),
        grid=(grid,),
        in_specs=[
            pl.BlockSpec((_BE, qw), lambda i: (i, 0)),
            pl.BlockSpec((_BE, HID), lambda i: (i, 0)),
            pl.BlockSpec((_BE, 1), lambda i: (i, 0)),
            pl.BlockSpec((_BE, 1), lambda i: (i, 0)),
            pl.BlockSpec((1, TD), lambda i: (0, 0)),
            pl.BlockSpec((1, TD), lambda i: (0, 0)),
            pl.BlockSpec((TD, HID), lambda i: (0, 0)),
        ],
        out_specs=[
            pl.BlockSpec((_BE, 8), lambda i: (i, 0)),
            pl.BlockSpec((1, 1, HEADS), lambda i: (i, 0, 0)),
        ],
        out_shape=[
            jax.ShapeDtypeStruct((EP, 8), jnp.float32),
            jax.ShapeDtypeStruct((grid, 1, HEADS), jnp.float32),
        ],
    )(qd, kvs, rt_col, t_col, bf.reshape(1, TD), ph.reshape(1, TD), we)


def _msg_body(ax_ref, gm_ref, v_ref, bf_ref, ph_ref, we_ref,
              lo_ref, hi_ref, wp_ref):
    ax = ax_ref[...]
    rel_t = ax[:, HEADS:HEADS + 1]
    em = jnp.dot(_enc(rel_t, bf_ref, ph_ref), we_ref[...],
                 preferred_element_type=jnp.float32)
    w = jnp.exp(ax[:, :HEADS] - gm_ref[...])
    wb = jnp.broadcast_to(w.reshape(_BE, HEADS, 1), (_BE, HEADS, C))
    msg = (v_ref[...] + em) * wb.reshape(_BE, HID)
    lo_ref[...] = msg[:, :SW]
    hi_ref[...] = msg[:, SW:]
    wp_ref[...] = jnp.concatenate(
        [w, jnp.zeros((_BE, SW - HEADS), jnp.float32)], axis=1)


def _msg_pass(ax, gmax, kvs, bf, ph, we):
    grid = EP // _BE
    return pl.pallas_call(
        _msg_body,
        grid=(grid,),
        in_specs=[
            pl.BlockSpec((_BE, 8), lambda i: (i, 0)),
            pl.BlockSpec((1, HEADS), lambda i: (0, 0)),
            pl.BlockSpec((_BE, HID), lambda i: (i, 1)),
            pl.BlockSpec((1, TD), lambda i: (0, 0)),
            pl.BlockSpec((1, TD), lambda i: (0, 0)),
            pl.BlockSpec((TD, HID), lambda i: (0, 0)),
        ],
        out_specs=[
            pl.BlockSpec((_BE, SW), lambda i: (i, 0)),
            pl.BlockSpec((_BE, SW), lambda i: (i, 0)),
            pl.BlockSpec((_BE, SW), lambda i: (i, 0)),
        ],
        out_shape=[
            jax.ShapeDtypeStruct((EP, SW), jnp.float32),
            jax.ShapeDtypeStruct((EP, SW), jnp.float32),
            jax.ShapeDtypeStruct((EP, SW), jnp.float32),
        ],
    )(ax, gmax, kvs, bf.reshape(1, TD), ph.reshape(1, TD), we)


def _hupd_body(lo_ref, hi_ref, wp_ref, h_ref, ws_ref, bs_ref, g_ref, be_ref,
               o_ref):
    lo = lo_ref[0]
    hi = hi_ref[0]
    blk = lo.shape[0]
    msg = jnp.concatenate([lo, hi], axis=1)
    den = wp_ref[0][:, :HEADS]
    den_b = jnp.broadcast_to(den.reshape(blk, HEADS, 1), (blk, HEADS, C))
    den_b = den_b.reshape(blk, HID)
    out = msg / jnp.maximum(den_b, 1e-30)
    out = out + jnp.dot(h_ref[...], ws_ref[...],
                        preferred_element_type=jnp.float32) + bs_ref[...]
    o_ref[...] = _bn(jax.nn.relu(out), g_ref[...], be_ref[...])


def _h_update_half(lo3, hi3, wp3, h, ws, bs, g, be, p):
    blk = 512
    k = QUART // blk
    acc_spec = pl.BlockSpec((1, blk, SW), lambda c, i: (c, i, 0))
    return pl.pallas_call(
        _hupd_body,
        grid=(NC, k),
        in_specs=[
            acc_spec, acc_spec, acc_spec,
            pl.BlockSpec((blk, HID), lambda c, i: (p * 2 * k + c * k + i, 0)),
            pl.BlockSpec((HID, HID), lambda c, i: (0, 0)),
            pl.BlockSpec((1, HID), lambda c, i: (0, 0)),
            pl.BlockSpec((1, HID), lambda c, i: (0, 0)),
            pl.BlockSpec((1, HID), lambda c, i: (0, 0)),
        ],
        out_specs=pl.BlockSpec((blk, HID), lambda c, i: (c * k + i, 0)),
        out_shape=jax.ShapeDtypeStruct((2 * QUART, HID), jnp.float32),
    )(lo3, hi3, wp3, h, ws, bs.reshape(1, HID), g.reshape(1, HID),
      be.reshape(1, HID))


def _head_body(h_ref, w1_ref, b1_ref, g1_ref, e1_ref, w2_ref, b2_ref,
               g2_ref, e2_ref, w3_ref, b3_ref, o_ref):
    z = jnp.dot(h_ref[...], w1_ref[...], preferred_element_type=jnp.float32)
    z = jax.nn.relu(_bn(z + b1_ref[...], g1_ref[...], e1_ref[...]))
    z = jnp.dot(z, w2_ref[...], preferred_element_type=jnp.float32)
    z = jax.nn.relu(_bn(z + b2_ref[...], g2_ref[...], e2_ref[...]))
    o_ref[...] = jnp.dot(z, w3_ref[...],
                         preferred_element_type=jnp.float32) + b3_ref[...]


def _head(h, p):
    blk = 512
    h2 = HID // 2
    return pl.pallas_call(
        _head_body,
        grid=(BS // blk,),
        in_specs=[
            pl.BlockSpec((blk, HID), lambda i: (i, 0)),
            pl.BlockSpec((HID, HID), lambda i: (0, 0)),
            pl.BlockSpec((1, HID), lambda i: (0, 0)),
            pl.BlockSpec((1, HID), lambda i: (0, 0)),
            pl.BlockSpec((1, HID), lambda i: (0, 0)),
            pl.BlockSpec((HID, h2), lambda i: (0, 0)),
            pl.BlockSpec((1, h2), lambda i: (0, 0)),
            pl.BlockSpec((1, h2), lambda i: (0, 0)),
            pl.BlockSpec((1, h2), lambda i: (0, 0)),
            pl.BlockSpec((h2, 1), lambda i: (0, 0)),
            pl.BlockSpec((1, 1), lambda i: (0, 0)),
        ],
        out_specs=pl.BlockSpec((blk, 1), lambda i: (i, 0)),
        out_shape=jax.ShapeDtypeStruct((BS, 1), jnp.float32),
    )(h, p['W1'], p['b1'].reshape(1, HID), p['g1'].reshape(1, HID),
      p['be1'].reshape(1, HID), p['W2'], p['b2'].reshape(1, h2),
      p['g2'].reshape(1, h2), p['be2'].reshape(1, h2),
      p['W3'], p['b3'].reshape(1, 1))


_MESH = plsc.VectorSubcoreMesh(core_axis_name="c", subcore_axis_name="s")
_SC_PARAMS = pltpu.CompilerParams(needs_layout_passes=False)


def _make_sc_gather(qw):
    @functools.partial(
        pl.kernel,
        out_type=[
            jax.ShapeDtypeStruct((EP, qw), jnp.float32),
            jax.ShapeDtypeStruct((EP, 2 * HID), jnp.float32),
        ],
        mesh=_MESH,
        scratch_types=[
            pltpu.VMEM((2, GB), jnp.int32),
            pltpu.VMEM((2, GB), jnp.int32),
            pltpu.VMEM((2, GB, qw), jnp.float32),
            pltpu.VMEM((2, GB, 2 * HID), jnp.float32),
            pltpu.SemaphoreType.DMA,
            pltpu.SemaphoreType.DMA,
            pltpu.SemaphoreType.DMA,
            pltpu.SemaphoreType.DMA,
            pltpu.SemaphoreType.DMA,
            pltpu.SemaphoreType.DMA,
        ],
        compiler_params=_SC_PARAMS,
    )
    def sc_gather(qt_hbm, kvt_hbm, dg_hbm, sg_hbm,
                  qd_out, kvs_out, di_v, si_v, qrows, kvrows,
                  isem0, isem1, gsem0, gsem1, wsem0, wsem1):
        wid = lax.axis_index("s") * NC + lax.axis_index("c")
        base = wid * PER_W
        ncheck = PER_W // GB
        njj = ncheck // 2
        isem = (isem0, isem1)
        gsem = (gsem0, gsem1)
        wsem = (wsem0, wsem1)

        def issue_idx(b, off):
            pltpu.async_copy(dg_hbm.at[pl.ds(off, GB)], di_v.at[b], isem[b])
            pltpu.async_copy(sg_hbm.at[pl.ds(off, GB)], si_v.at[b], isem[b])

        def wait_idx(b):
            pltpu.make_async_copy(dg_hbm.at[pl.ds(0, GB)], di_v.at[b],
                                  isem[b]).wait()
            pltpu.make_async_copy(sg_hbm.at[pl.ds(0, GB)], si_v.at[b],
                                  isem[b]).wait()

        def issue_gather(b):
            pltpu.async_copy(qt_hbm.at[di_v.at[b]], qrows.at[b], gsem[b])
            pltpu.async_copy(kvt_hbm.at[si_v.at[b]], kvrows.at[b], gsem[b])

        def wait_gather(b):
            pltpu.make_async_copy(qt_hbm.at[pl.ds(0, GB)], qrows.at[b],
                                  gsem[b]).wait()
            pltpu.make_async_copy(kvt_hbm.at[pl.ds(0, GB)], kvrows.at[b],
                                  gsem[b]).wait()

        def issue_wb(b, off):
            pltpu.async_copy(qrows.at[b], qd_out.at[pl.ds(off, GB)], wsem[b])
            pltpu.async_copy(kvrows.at[b], kvs_out.at[pl.ds(off, GB)], wsem[b])

        def wait_wb(b):
            pltpu.make_async_copy(qrows.at[b], qd_out.at[pl.ds(0, GB)],
                                  wsem[b]).wait()
            pltpu.make_async_copy(kvrows.at[b], kvs_out.at[pl.ds(0, GB)],
                                  wsem[b]).wait()

        issue_idx(0, base)

        def slot(jj, carry):
            for b in (0, 1):
                j2 = 2 * jj + b
                off = base + j2 * GB
                bp = 1 - b
                wait_idx(b)

                @pl.when(jj >= 1)
                def _():
                    wait_wb(b)

                issue_gather(b)
                if b == 1:
                    wait_gather(bp)
                    issue_wb(bp, off - GB)
                else:
                    @pl.when(jj >= 1)
                    def _():
                        wait_gather(bp)
                        issue_wb(bp, off - GB)
                if b == 0:
                    issue_idx(bp, off + GB)
                else:
                    @pl.when(jj < njj - 1)
                    def _():
                        issue_idx(bp, off + GB)
            return carry

        lax.fori_loop(0, njj, slot, 0)
        wait_gather(1)
        issue_wb(1, base + (ncheck - 1) * GB)
        wait_wb(0)
        wait_wb(1)

    return sc_gather


_sc_gather0 = _make_sc_gather(QW0)
_sc_gather1 = _make_sc_gather(QW1)


def _make_sc_scatter(p):
    acc_t = jax.ShapeDtypeStruct((NC, ACC_ROWS, SW), jnp.float32)

    @functools.partial(
        pl.kernel,
        out_type=[acc_t, acc_t, acc_t],
        mesh=_MESH,
        scratch_types=[
            pltpu.VMEM((2, SB), jnp.int32),
            pltpu.VMEM((2, SB), jnp.int32),
            pltpu.VMEM((2, SB, SW), jnp.float32),
            pltpu.VMEM((2, SB, SW), jnp.float32),
            pltpu.VMEM((2, SB, SW), jnp.float32),
            pltpu.VMEM_SHARED((ACC_ROWS, SW), jnp.float32),
            pltpu.VMEM_SHARED((ACC_ROWS, SW), jnp.float32),
            pltpu.VMEM_SHARED((ACC_ROWS, SW), jnp.float32),
            pltpu.SemaphoreType.DMA,
            pltpu.SemaphoreType.DMA,
        ],
        compiler_params=_SC_PARAMS,
    )
    def sc_scatter(lo_hbm, hi_hbm, wp_hbm, ds_hbm, zrows_hbm,
                   lo_out, hi_out, wp_out,
                   di_v, ai_v, lo_v, hi_v, wp_v, lo_sh, hi_sh, wp_sh,
                   lsem0, lsem1):
        cid = lax.axis_index("c")
        sid = lax.axis_index("s")
        nbase = (2 * p + cid) * QUART
        ncheck = PER_T // SB
        njj = ncheck // 2
        lsem = (lsem0, lsem1)

        zslice = pl.ds(sid * DRAIN, DRAIN)
        pltpu.sync_copy(zrows_hbm, lo_sh.at[zslice])
        pltpu.sync_copy(zrows_hbm, hi_sh.at[zslice])
        pltpu.sync_copy(zrows_hbm, wp_sh.at[zslice])
        plsc.subcore_barrier()

        def issue_loads(b, off):
            sl_rows = pl.ds(off, SB)
            pltpu.async_copy(ds_hbm.at[sl_rows], di_v.at[b], lsem[b])
            pltpu.async_copy(lo_hbm.at[sl_rows], lo_v.at[b], lsem[b])
            pltpu.async_copy(hi_hbm.at[sl_rows], hi_v.at[b], lsem[b])
            pltpu.async_copy(wp_hbm.at[sl_rows], wp_v.at[b], lsem[b])

        def wait_loads(b):
            sl0 = pl.ds(0, SB)
            pltpu.make_async_copy(ds_hbm.at[sl0], di_v.at[b], lsem[b]).wait()
            pltpu.make_async_copy(lo_hbm.at[sl0], lo_v.at[b], lsem[b]).wait()
            pltpu.make_async_copy(hi_hbm.at[sl0], hi_v.at[b], lsem[b]).wait()
            pltpu.make_async_copy(wp_hbm.at[sl0], wp_v.at[b], lsem[b]).wait()

        base_t = sid * PER_T
        issue_loads(0, base_t)

        def slot(jj, carry):
            for b in (0, 1):
                j2 = 2 * jj + b
                off = base_t + j2 * SB
                bp = 1 - b
                wait_loads(b)
                for k in range(SB // 16):
                    sl = pl.ds(k * 16, 16)
                    rel = di_v[b, sl] - nbase
                    ok = (rel >= 0) & (rel < QUART)
                    ai_v[b, sl] = jnp.where(ok, rel, QUART)
                if b == 0:
                    issue_loads(bp, off + SB)
                else:
                    @pl.when(jj < njj - 1)
                    def _():
                        issue_loads(bp, off + SB)
                pltpu.sync_copy(lo_v.at[b], lo_sh.at[ai_v.at[b]], add=True)
                pltpu.sync_copy(hi_v.at[b], hi_sh.at[ai_v.at[b]], add=True)
                pltpu.sync_copy(wp_v.at[b], wp_sh.at[ai_v.at[b]], add=True)
            return carry

        lax.fori_loop(0, njj, slot, 0)
        plsc.subcore_barrier()
        pltpu.sync_copy(lo_sh.at[zslice], lo_out.at[cid, zslice])
        pltpu.sync_copy(hi_sh.at[zslice], hi_out.at[cid, zslice])
        pltpu.sync_copy(wp_sh.at[zslice], wp_out.at[cid, zslice])

    return sc_scatter


_sc_scatter0 = _make_sc_scatter(0)
_sc_scatter1 = _make_sc_scatter(1)


def kernel(x, edge_index, time, node_time, batch_size, params):
    src = edge_index[0]
    dst = edge_index[1]
    pad = EP - E
    dg = jnp.concatenate([dst, jnp.zeros((pad,), jnp.int32)])
    sg = jnp.concatenate([src, jnp.zeros((pad,), jnp.int32)])
    ds_ = jnp.concatenate([dst, jnp.full((pad,), jnp.int32(2 ** 20))])
    t_col = jnp.concatenate([time, jnp.zeros((pad,), jnp.float32)]).reshape(EP, 1)
    nt_col = node_time.reshape(N, 1)
    zrows = jnp.zeros((DRAIN, SW), jnp.float32)

    p = params
    h = _input_proj(x, p['W_in'], p['b_in'])
    rt_col = t_col
    for l in range(NL):
        qw = QW0 if l == 0 else QW1
        qt, kvt = _projections(h, nt_col, p['Wq'][l], p['bq'][l],
                               p['Wk'][l], p['bk'][l], p['Wv'][l], p['bv'][l],
                               qw)
        gather = _sc_gather0 if l == 0 else _sc_gather1
        qd, kvs = gather(qt, kvt, dg, sg)
        ax, bmax = _alpha_pass(qd, kvs, rt_col, t_col,
                               p['basis_freq'], p['phase'], p['We'][l], qw)
        rt_col = ax[:, HEADS:HEADS + 1]
        gmax = jnp.max(bmax[:, 0, :], axis=0, keepdims=True)
        lo, hi, wp = _msg_pass(ax, gmax, kvs, p['basis_freq'], p['phase'],
                               p['We'][l])
        lo_a, hi_a, wp_a = _sc_scatter0(lo, hi, wp, ds_, zrows)
        lo_b, hi_b, wp_b = _sc_scatter1(lo, hi, wp, ds_, zrows)
        h_a = _h_update_half(lo_a, hi_a, wp_a, h, p['Wskip'][l], p['bskip'][l],
                             p['gamma'][l], p['beta'][l], 0)
        h_b = _h_update_half(lo_b, hi_b, wp_b, h, p['Wskip'][l], p['bskip'][l],
                             p['gamma'][l], p['beta'][l], 1)
        h = jnp.concatenate([h_a, h_b[:N - 2 * QUART]], axis=0)
    z = _head(h, p)
    return z[:, 0]


# speedup vs baseline: 1.7608x; 1.1900x over previous
import functools

import jax
import jax.numpy as jnp
from jax import lax
from jax.experimental import pallas as pl
from jax.experimental.pallas import tpu as pltpu
from jax.experimental.pallas import tpu_sc as plsc

N = 10000
E = 160000
IN = 128
HID = 256
HEADS = 4
C = HID // HEADS
TD = 64
NL = 2
BS = 4096

NC = 2
NS = 16
NW = NC * NS
EP = 163840
PER_W = EP // NW
GB = 64
PER_T = EP // NS
SB = 64
QW0 = 384
QW1 = 256
SW = 128
HALF = 5000
ACC_ROWS = 5120
DRAIN = ACC_ROWS // NS
NP = 2 * ACC_ROWS

_EPS_BN = 1e-5
_BN_SCALE = 1.0 / (1.0 + _EPS_BN) ** 0.5


def _bn(h, g, b):
    return h * (g * _BN_SCALE) + b


def _inproj_body(x_ref, w_ref, b_ref, o_ref):
    o_ref[...] = jax.nn.relu(
        jnp.dot(x_ref[...], w_ref[...], preferred_element_type=jnp.float32)
        + b_ref[...])


def _input_proj(x, w, b):
    blk = 2000
    return pl.pallas_call(
        _inproj_body,
        grid=(N // blk,),
        in_specs=[
            pl.BlockSpec((blk, IN), lambda i: (i, 0)),
            pl.BlockSpec((IN, HID), lambda i: (0, 0)),
            pl.BlockSpec((1, HID), lambda i: (0, 0)),
        ],
        out_specs=pl.BlockSpec((blk, HID), lambda i: (i, 0)),
        out_shape=jax.ShapeDtypeStruct((N, HID), jnp.float32),
    )(x, w, b.reshape(1, HID))


def _make_proj_body(qw):
    def body(h_ref, nt_ref, wq_ref, bq_ref, wk_ref, bk_ref, wv_ref, bv_ref,
             qt_ref, kvt_ref):
        h = h_ref[...]
        q = jnp.dot(h, wq_ref[...], preferred_element_type=jnp.float32) + bq_ref[...]
        k = jnp.dot(h, wk_ref[...], preferred_element_type=jnp.float32) + bk_ref[...]
        v = jnp.dot(h, wv_ref[...], preferred_element_type=jnp.float32) + bv_ref[...]
        if qw > HID:
            pad = jnp.zeros((h.shape[0], qw - HID - 1), jnp.float32)
            qt_ref[...] = jnp.concatenate([q, nt_ref[...], pad], axis=1)
        else:
            qt_ref[...] = q
        kvt_ref[...] = jnp.concatenate([k, v], axis=1)
    return body


def _projections(h, nt, wq, bq, wk, bk, wv, bv, qw):
    blk = 1024
    w_spec = pl.BlockSpec((HID, HID), lambda i: (0, 0))
    b_spec = pl.BlockSpec((1, HID), lambda i: (0, 0))
    return pl.pallas_call(
        _make_proj_body(qw),
        grid=(NP // blk,),
        in_specs=[
            pl.BlockSpec((blk, HID), lambda i: (i, 0)),
            pl.BlockSpec((blk, 1), lambda i: (i, 0)),
            w_spec, b_spec, w_spec, b_spec, w_spec, b_spec,
        ],
        out_specs=[
            pl.BlockSpec((blk, qw), lambda i: (i, 0)),
            pl.BlockSpec((blk, 2 * HID), lambda i: (i, 0)),
        ],
        out_shape=[
            jax.ShapeDtypeStruct((NP, qw), jnp.float32),
            jax.ShapeDtypeStruct((NP, 2 * HID), jnp.float32),
        ],
    )(h, nt, wq, bq.reshape(1, HID), wk, bk.reshape(1, HID),
      wv, bv.reshape(1, HID))


_BE = 2048


def _enc(rel_t, bf_ref, ph_ref):
    return jnp.cos(rel_t * bf_ref[...] + ph_ref[...])


def _make_alpha_body(qw):
    def body(qd_ref, k_ref, rt_ref, t_ref, bf_ref, ph_ref, we_ref,
             ax_ref, bm_ref):
        qd = qd_ref[...]
        if qw > HID:
            rel_t = qd[:, HID:HID + 1] - t_ref[...]
        else:
            rel_t = rt_ref[...]
        em = jnp.dot(_enc(rel_t, bf_ref, ph_ref), we_ref[...],
                     preferred_element_type=jnp.float32)
        kk = k_ref[...] + em
        prod = (qd[:, :HID] * kk).reshape(_BE, HEADS, C)
        scale = 1.0 / (C ** 0.5)
        alpha = prod.sum(axis=-1) * scale
        pad = jnp.zeros((_BE, 3), jnp.float32)
        ax_ref[...] = jnp.concatenate([alpha, rel_t, pad], axis=1)
        bm_ref[...] = jnp.max(alpha, axis=0).reshape(1, 1, HEADS)
    return body


def _alpha_pass(qd, kvs, rt_col, t_col, bf, ph, we, qw):
    grid = EP // _BE
    return pl.pallas_call(
        _make_alpha_body(qw),
        grid=(grid,),
        in_specs=[
            pl.BlockSpec((_BE, qw), lambda i: (i, 0)),
            pl.BlockSpec((_BE, HID), lambda i: (i, 0)),
            pl.BlockSpec((_BE, 1), lambda i: (i, 0)),
            pl.BlockSpec((_BE, 1), lambda i: (i, 0)),
            pl.BlockSpec((1, TD), lambda i: (0, 0)),
            pl.BlockSpec((1, TD), lambda i: (0, 0)),
            pl.BlockSpec((TD, HID), lambda i: (0, 0)),
        ],
        out_specs=[
            pl.BlockSpec((_BE, 8), lambda i: (i, 0)),
            pl.BlockSpec((1, 1, HEADS), lambda i: (i, 0, 0)),
        ],
        out_shape=[
            jax.ShapeDtypeStruct((EP, 8), jnp.float32),
            jax.ShapeDtypeStruct((grid, 1, HEADS), jnp.float32),
        ],
    )(qd, kvs, rt_col, t_col, bf.reshape(1, TD), ph.reshape(1, TD), we)


def _msg_body(ax_ref, gm_ref, v_ref, bf_ref, ph_ref, we_ref,
              lo_ref, hi_ref, wp_ref):
    ax = ax_ref[...]
    rel_t = ax[:, HEADS:HEADS + 1]
    em = jnp.dot(_enc(rel_t, bf_ref, ph_ref), we_ref[...],
                 preferred_element_type=jnp.float32)
    w = jnp.exp(ax[:, :HEADS] - gm_ref[...])
    wb = jnp.broadcast_to(w.reshape(_BE, HEADS, 1), (_BE, HEADS, C))
    msg = (v_ref[...] + em) * wb.reshape(_BE, HID)
    lo_ref[...] = msg[:, :SW]
    hi_ref[...] = msg[:, SW:]
    wp_ref[...] = jnp.concatenate(
        [w, jnp.zeros((_BE, SW - HEADS), jnp.float32)], axis=1)


def _msg_pass(ax, gmax, kvs, bf, ph, we):
    grid = EP // _BE
    return pl.pallas_call(
        _msg_body,
        grid=(grid,),
        in_specs=[
            pl.BlockSpec((_BE, 8), lambda i: (i, 0)),
            pl.BlockSpec((1, HEADS), lambda i: (0, 0)),
            pl.BlockSpec((_BE, HID), lambda i: (i, 1)),
            pl.BlockSpec((1, TD), lambda i: (0, 0)),
            pl.BlockSpec((1, TD), lambda i: (0, 0)),
            pl.BlockSpec((TD, HID), lambda i: (0, 0)),
        ],
        out_specs=[
            pl.BlockSpec((_BE, SW), lambda i: (i, 0)),
            pl.BlockSpec((_BE, SW), lambda i: (i, 0)),
            pl.BlockSpec((_BE, SW), lambda i: (i, 0)),
        ],
        out_shape=[
            jax.ShapeDtypeStruct((EP, SW), jnp.float32),
            jax.ShapeDtypeStruct((EP, SW), jnp.float32),
            jax.ShapeDtypeStruct((EP, SW), jnp.float32),
        ],
    )(ax, gmax, kvs, bf.reshape(1, TD), ph.reshape(1, TD), we)


def _hupd_body(lo_ref, hi_ref, wp_ref, h_ref, ws_ref, bs_ref, g_ref, be_ref,
               o_ref):
    lo = lo_ref[0]
    hi = hi_ref[0]
    blk = lo.shape[0]
    msg = jnp.concatenate([lo, hi], axis=1)
    den = wp_ref[0][:, :HEADS]
    den_b = jnp.broadcast_to(den.reshape(blk, HEADS, 1), (blk, HEADS, C))
    den_b = den_b.reshape(blk, HID)
    out = msg / jnp.maximum(den_b, 1e-30)
    out = out + jnp.dot(h_ref[...], ws_ref[...],
                        preferred_element_type=jnp.float32) + bs_ref[...]
    o_ref[...] = _bn(jax.nn.relu(out), g_ref[...], be_ref[...])


def _h_update(lo3, hi3, wp3, h, ws, bs, g, be):
    blk = 512
    k = ACC_ROWS // blk
    acc_spec = pl.BlockSpec((1, blk, SW), lambda c, i: (c, i, 0))
    return pl.pallas_call(
        _hupd_body,
        grid=(NC, k),
        in_specs=[
            acc_spec, acc_spec, acc_spec,
            pl.BlockSpec((blk, HID), lambda c, i: (c * k + i, 0)),
            pl.BlockSpec((HID, HID), lambda c, i: (0, 0)),
            pl.BlockSpec((1, HID), lambda c, i: (0, 0)),
            pl.BlockSpec((1, HID), lambda c, i: (0, 0)),
            pl.BlockSpec((1, HID), lambda c, i: (0, 0)),
        ],
        out_specs=pl.BlockSpec((blk, HID), lambda c, i: (c * k + i, 0)),
        out_shape=jax.ShapeDtypeStruct((NP, HID), jnp.float32),
    )(lo3, hi3, wp3, h, ws, bs.reshape(1, HID), g.reshape(1, HID),
      be.reshape(1, HID))


def _head_body(h_ref, w1_ref, b1_ref, g1_ref, e1_ref, w2_ref, b2_ref,
               g2_ref, e2_ref, w3_ref, b3_ref, o_ref):
    z = jnp.dot(h_ref[...], w1_ref[...], preferred_element_type=jnp.float32)
    z = jax.nn.relu(_bn(z + b1_ref[...], g1_ref[...], e1_ref[...]))
    z = jnp.dot(z, w2_ref[...], preferred_element_type=jnp.float32)
    z = jax.nn.relu(_bn(z + b2_ref[...], g2_ref[...], e2_ref[...]))
    o_ref[...] = jnp.dot(z, w3_ref[...],
                         preferred_element_type=jnp.float32) + b3_ref[...]


def _head(h, p):
    blk = 512
    h2 = HID // 2
    return pl.pallas_call(
        _head_body,
        grid=(BS // blk,),
        in_specs=[
            pl.BlockSpec((blk, HID), lambda i: (i, 0)),
            pl.BlockSpec((HID, HID), lambda i: (0, 0)),
            pl.BlockSpec((1, HID), lambda i: (0, 0)),
            pl.BlockSpec((1, HID), lambda i: (0, 0)),
            pl.BlockSpec((1, HID), lambda i: (0, 0)),
            pl.BlockSpec((HID, h2), lambda i: (0, 0)),
            pl.BlockSpec((1, h2), lambda i: (0, 0)),
            pl.BlockSpec((1, h2), lambda i: (0, 0)),
            pl.BlockSpec((1, h2), lambda i: (0, 0)),
            pl.BlockSpec((h2, 1), lambda i: (0, 0)),
            pl.BlockSpec((1, 1), lambda i: (0, 0)),
        ],
        out_specs=pl.BlockSpec((blk, 1), lambda i: (i, 0)),
        out_shape=jax.ShapeDtypeStruct((BS, 1), jnp.float32),
    )(h, p['W1'], p['b1'].reshape(1, HID), p['g1'].reshape(1, HID),
      p['be1'].reshape(1, HID), p['W2'], p['b2'].reshape(1, h2),
      p['g2'].reshape(1, h2), p['be2'].reshape(1, h2),
      p['W3'], p['b3'].reshape(1, 1))


_MESH = plsc.VectorSubcoreMesh(core_axis_name="c", subcore_axis_name="s")
_SC_PARAMS = pltpu.CompilerParams(needs_layout_passes=False)


def _make_sc_gather(qw):
    @functools.partial(
        pl.kernel,
        out_type=[
            jax.ShapeDtypeStruct((EP, qw), jnp.float32),
            jax.ShapeDtypeStruct((EP, 2 * HID), jnp.float32),
        ],
        mesh=_MESH,
        scratch_types=[
            pltpu.VMEM((2, GB), jnp.int32),
            pltpu.VMEM((2, GB), jnp.int32),
            pltpu.VMEM((2, GB, qw), jnp.float32),
            pltpu.VMEM((2, GB, 2 * HID), jnp.float32),
            pltpu.SemaphoreType.DMA,
            pltpu.SemaphoreType.DMA,
            pltpu.SemaphoreType.DMA,
            pltpu.SemaphoreType.DMA,
            pltpu.SemaphoreType.DMA,
            pltpu.SemaphoreType.DMA,
        ],
        compiler_params=_SC_PARAMS,
    )
    def sc_gather(qt_hbm, kvt_hbm, dg_hbm, sg_hbm,
                  qd_out, kvs_out, di_v, si_v, qrows, kvrows,
                  isem0, isem1, gsem0, gsem1, wsem0, wsem1):
        wid = lax.axis_index("s") * NC + lax.axis_index("c")
        base = wid * PER_W
        ncheck = PER_W // GB
        njj = ncheck // 2
        isem = (isem0, isem1)
        gsem = (gsem0, gsem1)
        wsem = (wsem0, wsem1)

        def issue_idx(b, off):
            pltpu.async_copy(dg_hbm.at[pl.ds(off, GB)], di_v.at[b], isem[b])
            pltpu.async_copy(sg_hbm.at[pl.ds(off, GB)], si_v.at[b], isem[b])

        def wait_idx(b):
            pltpu.make_async_copy(dg_hbm.at[pl.ds(0, GB)], di_v.at[b],
                                  isem[b]).wait()
            pltpu.make_async_copy(sg_hbm.at[pl.ds(0, GB)], si_v.at[b],
                                  isem[b]).wait()

        def issue_gather(b):
            pltpu.async_copy(qt_hbm.at[di_v.at[b]], qrows.at[b], gsem[b])
            pltpu.async_copy(kvt_hbm.at[si_v.at[b]], kvrows.at[b], gsem[b])

        def wait_gather(b):
            pltpu.make_async_copy(qt_hbm.at[pl.ds(0, GB)], qrows.at[b],
                                  gsem[b]).wait()
            pltpu.make_async_copy(kvt_hbm.at[pl.ds(0, GB)], kvrows.at[b],
                                  gsem[b]).wait()

        def issue_wb(b, off):
            pltpu.async_copy(qrows.at[b], qd_out.at[pl.ds(off, GB)], wsem[b])
            pltpu.async_copy(kvrows.at[b], kvs_out.at[pl.ds(off, GB)], wsem[b])

        def wait_wb(b):
            pltpu.make_async_copy(qrows.at[b], qd_out.at[pl.ds(0, GB)],
                                  wsem[b]).wait()
            pltpu.make_async_copy(kvrows.at[b], kvs_out.at[pl.ds(0, GB)],
                                  wsem[b]).wait()

        issue_idx(0, base)

        def slot(jj, carry):
            for b in (0, 1):
                j2 = 2 * jj + b
                off = base + j2 * GB
                bp = 1 - b
                wait_idx(b)

                @pl.when(jj >= 1)
                def _():
                    wait_wb(b)

                issue_gather(b)
                if b == 1:
                    wait_gather(bp)
                    issue_wb(bp, off - GB)
                else:
                    @pl.when(jj >= 1)
                    def _():
                        wait_gather(bp)
                        issue_wb(bp, off - GB)
                if b == 0:
                    issue_idx(bp, off + GB)
                else:
                    @pl.when(jj < njj - 1)
                    def _():
                        issue_idx(bp, off + GB)
            return carry

        lax.fori_loop(0, njj, slot, 0)
        wait_gather(1)
        issue_wb(1, base + (ncheck - 1) * GB)
        wait_wb(0)
        wait_wb(1)

    return sc_gather


_sc_gather0 = _make_sc_gather(QW0)
_sc_gather1 = _make_sc_gather(QW1)


def _make_sc_scatter(narr):
    acc_t = jax.ShapeDtypeStruct((NC, ACC_ROWS, SW), jnp.float32)
    buf_t = pltpu.VMEM((2, SB, SW), jnp.float32)
    sh_t = pltpu.VMEM_SHARED((ACC_ROWS, SW), jnp.float32)

    @functools.partial(
        pl.kernel,
        out_type=[acc_t] * narr,
        mesh=_MESH,
        scratch_types=(
            [pltpu.VMEM((2, SB), jnp.int32), pltpu.VMEM((2, SB), jnp.int32)]
            + [buf_t] * narr + [sh_t] * narr
            + [pltpu.SemaphoreType.DMA, pltpu.SemaphoreType.DMA]
        ),
        compiler_params=_SC_PARAMS,
    )
    def sc_scatter(*refs):
        data_hbm = refs[:narr]
        ds_hbm = refs[narr]
        zrows_hbm = refs[narr + 1]
        outs = refs[narr + 2:2 * narr + 2]
        di_v = refs[2 * narr + 2]
        ai_v = refs[2 * narr + 3]
        bufs = refs[2 * narr + 4:3 * narr + 4]
        shs = refs[3 * narr + 4:4 * narr + 4]
        lsem = refs[4 * narr + 4:4 * narr + 6]
        cid = lax.axis_index("c")
        sid = lax.axis_index("s")
        nbase = cid * HALF
        ncheck = PER_T // SB
        njj = ncheck // 2

        zslice = pl.ds(sid * DRAIN, DRAIN)
        for sh in shs:
            pltpu.sync_copy(zrows_hbm, sh.at[zslice])
        plsc.subcore_barrier()

        def issue_loads(b, off):
            sl_rows = pl.ds(off, SB)
            pltpu.async_copy(ds_hbm.at[sl_rows], di_v.at[b], lsem[b])
            for src, buf in zip(data_hbm, bufs):
                pltpu.async_copy(src.at[sl_rows], buf.at[b], lsem[b])

        def wait_loads(b):
            sl0 = pl.ds(0, SB)
            pltpu.make_async_copy(ds_hbm.at[sl0], di_v.at[b], lsem[b]).wait()
            for src, buf in zip(data_hbm, bufs):
                pltpu.make_async_copy(src.at[sl0], buf.at[b], lsem[b]).wait()

        base_t = sid * PER_T
        issue_loads(0, base_t)

        def slot(jj, carry):
            for b in (0, 1):
                j2 = 2 * jj + b
                off = base_t + j2 * SB
                bp = 1 - b
                wait_loads(b)
                for k in range(SB // 16):
                    sl = pl.ds(k * 16, 16)
                    rel = di_v[b, sl] - nbase
                    ok = (rel >= 0) & (rel < HALF)
                    ai_v[b, sl] = jnp.where(ok, rel, HALF)
                if b == 0:
                    issue_loads(bp, off + SB)
                else:
                    @pl.when(jj < njj - 1)
                    def _():
                        issue_loads(bp, off + SB)
                for buf, sh in zip(bufs, shs):
                    pltpu.sync_copy(buf.at[b], sh.at[ai_v.at[b]], add=True)
            return carry

        lax.fori_loop(0, njj, slot, 0)
        plsc.subcore_barrier()
        for sh, out in zip(shs, outs):
            pltpu.sync_copy(sh.at[zslice], out.at[cid, zslice])

    return sc_scatter


_sc_scatter_mh = _make_sc_scatter(2)
_sc_scatter_w = _make_sc_scatter(1)


def kernel(x, edge_index, time, node_time, batch_size, params):
    src = edge_index[0]
    dst = edge_index[1]
    pad = EP - E
    pad_seg = ACC_ROWS - HALF
    dg = jnp.concatenate([dst, jnp.zeros((pad,), jnp.int32)])
    sg = jnp.concatenate([src, jnp.zeros((pad,), jnp.int32)])
    dgp = dg + pad_seg * (dg // HALF)
    sgp = sg + pad_seg * (sg // HALF)
    ds_ = jnp.concatenate([dst, jnp.full((pad,), jnp.int32(2 ** 20))])
    t_col = jnp.concatenate([time, jnp.zeros((pad,), jnp.float32)]).reshape(EP, 1)
    zseg = jnp.zeros((pad_seg,), jnp.float32)
    nt_col = jnp.concatenate(
        [node_time[:HALF], zseg, node_time[HALF:], zseg]).reshape(NP, 1)
    zrows = jnp.zeros((DRAIN, SW), jnp.float32)

    p = params
    h0 = _input_proj(x, p['W_in'], p['b_in'])
    zpad = jnp.zeros((pad_seg, HID), jnp.float32)
    h = jnp.concatenate([h0[:HALF], zpad, h0[HALF:], zpad], axis=0)
    rt_col = t_col
    for l in range(NL):
        qw = QW0 if l == 0 else QW1
        qt, kvt = _projections(h, nt_col, p['Wq'][l], p['bq'][l],
                               p['Wk'][l], p['bk'][l], p['Wv'][l], p['bv'][l],
                               qw)
        gather = _sc_gather0 if l == 0 else _sc_gather1
        qd, kvs = gather(qt, kvt, dgp, sgp)
        ax, bmax = _alpha_pass(qd, kvs, rt_col, t_col,
                               p['basis_freq'], p['phase'], p['We'][l], qw)
        rt_col = ax[:, HEADS:HEADS + 1]
        gmax = jnp.max(bmax[:, 0, :], axis=0, keepdims=True)
        lo, hi, wp = _msg_pass(ax, gmax, kvs, p['basis_freq'], p['phase'],
                               p['We'][l])
        lo3, hi3 = _sc_scatter_mh(lo, hi, ds_, zrows)
        wp3, = _sc_scatter_w(wp, ds_, zrows)
        h = _h_update(lo3, hi3, wp3, h, p['Wskip'][l], p['bskip'][l],
                      p['gamma'][l], p['beta'][l])
    z = _head(h, p)
    return z[:, 0]


# speedup vs baseline: 2.2862x; 1.2984x over previous
import functools

import jax
import jax.numpy as jnp
from jax import lax
from jax.experimental import pallas as pl
from jax.experimental.pallas import tpu as pltpu
from jax.experimental.pallas import tpu_sc as plsc

N = 10000
E = 160000
IN = 128
HID = 256
HEADS = 4
C = HID // HEADS
TD = 64
NL = 2
BS = 4096

NC = 2
NS = 16
NW = NC * NS
EP = 163840
PER_W = EP // NW
GB = 64
PER_T = EP // NS
SB = 64
QW0 = 384
QW1 = 256
SW = 128
HALF = 5000
ACC_ROWS = 5120
DRAIN = ACC_ROWS // NS
NP = 2 * ACC_ROWS

_EPS_BN = 1e-5
_BN_SCALE = 1.0 / (1.0 + _EPS_BN) ** 0.5


def _bn(h, g, b):
    return h * (g * _BN_SCALE) + b


def _inproj_body(x_ref, w_ref, b_ref, o_ref):
    o_ref[...] = jax.nn.relu(
        jnp.dot(x_ref[...], w_ref[...], preferred_element_type=jnp.float32)
        + b_ref[...])


def _input_proj(x, w, b):
    blk = 2000
    return pl.pallas_call(
        _inproj_body,
        grid=(N // blk,),
        in_specs=[
            pl.BlockSpec((blk, IN), lambda i: (i, 0)),
            pl.BlockSpec((IN, HID), lambda i: (0, 0)),
            pl.BlockSpec((1, HID), lambda i: (0, 0)),
        ],
        out_specs=pl.BlockSpec((blk, HID), lambda i: (i, 0)),
        out_shape=jax.ShapeDtypeStruct((N, HID), jnp.float32),
    )(x, w, b.reshape(1, HID))


def _make_proj_body(qw):
    def body(h_ref, nt_ref, wq_ref, bq_ref, wk_ref, bk_ref, wv_ref, bv_ref,
             qt_ref, kvt_ref):
        h = h_ref[...]
        q = jnp.dot(h, wq_ref[...], preferred_element_type=jnp.float32) + bq_ref[...]
        k = jnp.dot(h, wk_ref[...], preferred_element_type=jnp.float32) + bk_ref[...]
        v = jnp.dot(h, wv_ref[...], preferred_element_type=jnp.float32) + bv_ref[...]
        if qw > HID:
            pad = jnp.zeros((h.shape[0], qw - HID - 1), jnp.float32)
            qt_ref[...] = jnp.concatenate([q, nt_ref[...], pad], axis=1)
        else:
            qt_ref[...] = q
        kvt_ref[...] = jnp.concatenate([k, v], axis=1)
    return body


def _projections(h, nt, wq, bq, wk, bk, wv, bv, qw):
    blk = 1024
    w_spec = pl.BlockSpec((HID, HID), lambda i: (0, 0))
    b_spec = pl.BlockSpec((1, HID), lambda i: (0, 0))
    return pl.pallas_call(
        _make_proj_body(qw),
        grid=(NP // blk,),
        in_specs=[
            pl.BlockSpec((blk, HID), lambda i: (i, 0)),
            pl.BlockSpec((blk, 1), lambda i: (i, 0)),
            w_spec, b_spec, w_spec, b_spec, w_spec, b_spec,
        ],
        out_specs=[
            pl.BlockSpec((blk, qw), lambda i: (i, 0)),
            pl.BlockSpec((blk, 2 * HID), lambda i: (i, 0)),
        ],
        out_shape=[
            jax.ShapeDtypeStruct((NP, qw), jnp.float32),
            jax.ShapeDtypeStruct((NP, 2 * HID), jnp.float32),
        ],
    )(h, nt, wq, bq.reshape(1, HID), wk, bk.reshape(1, HID),
      wv, bv.reshape(1, HID))


_BE = 2048


def _enc(rel_t, bf_ref, ph_ref):
    return jnp.cos(rel_t * bf_ref[...] + ph_ref[...])


_ASHIFT = 30.0


def _make_edge_body(qw):
    def body(qd_ref, kv_ref, rt_ref, t_ref, bf_ref, ph_ref, we_ref,
             lo_ref, hi_ref, wp_ref, rt_out_ref):
        qd = qd_ref[...]
        if qw > HID:
            rel_t = qd[:, HID:HID + 1] - t_ref[...]
        else:
            rel_t = rt_ref[...]
        em = jnp.dot(_enc(rel_t, bf_ref, ph_ref), we_ref[...],
                     preferred_element_type=jnp.float32)
        kv = kv_ref[...]
        kk = kv[:, :HID] + em
        prod = (qd[:, :HID] * kk).reshape(_BE, HEADS, C)
        scale = 1.0 / (C ** 0.5)
        alpha = prod.sum(axis=-1) * scale
        w = jnp.exp(alpha - _ASHIFT)
        wb = jnp.broadcast_to(w.reshape(_BE, HEADS, 1), (_BE, HEADS, C))
        msg = (kv[:, HID:] + em) * wb.reshape(_BE, HID)
        lo_ref[...] = msg[:, :SW]
        hi_ref[...] = msg[:, SW:]
        wp_ref[...] = jnp.concatenate(
            [w, jnp.zeros((_BE, SW - HEADS), jnp.float32)], axis=1)
        rt_out_ref[...] = rel_t
    return body


def _edge_pass(qd, kvs, rt_col, t_col, bf, ph, we, qw):
    grid = EP // _BE
    return pl.pallas_call(
        _make_edge_body(qw),
        grid=(grid,),
        in_specs=[
            pl.BlockSpec((_BE, qw), lambda i: (i, 0)),
            pl.BlockSpec((_BE, 2 * HID), lambda i: (i, 0)),
            pl.BlockSpec((_BE, 1), lambda i: (i, 0)),
            pl.BlockSpec((_BE, 1), lambda i: (i, 0)),
            pl.BlockSpec((1, TD), lambda i: (0, 0)),
            pl.BlockSpec((1, TD), lambda i: (0, 0)),
            pl.BlockSpec((TD, HID), lambda i: (0, 0)),
        ],
        out_specs=[
            pl.BlockSpec((_BE, SW), lambda i: (i, 0)),
            pl.BlockSpec((_BE, SW), lambda i: (i, 0)),
            pl.BlockSpec((_BE, SW), lambda i: (i, 0)),
            pl.BlockSpec((_BE, 1), lambda i: (i, 0)),
        ],
        out_shape=[
            jax.ShapeDtypeStruct((EP, SW), jnp.float32),
            jax.ShapeDtypeStruct((EP, SW), jnp.float32),
            jax.ShapeDtypeStruct((EP, SW), jnp.float32),
            jax.ShapeDtypeStruct((EP, 1), jnp.float32),
        ],
    )(qd, kvs, rt_col, t_col, bf.reshape(1, TD), ph.reshape(1, TD), we)


def _hupd_body(lo_ref, hi_ref, wp_ref, h_ref, ws_ref, bs_ref, g_ref, be_ref,
               o_ref):
    lo = lo_ref[0]
    hi = hi_ref[0]
    blk = lo.shape[0]
    msg = jnp.concatenate([lo, hi], axis=1)
    den = wp_ref[0][:, :HEADS]
    den_b = jnp.broadcast_to(den.reshape(blk, HEADS, 1), (blk, HEADS, C))
    den_b = den_b.reshape(blk, HID)
    out = msg / jnp.maximum(den_b, 1e-30)
    out = out + jnp.dot(h_ref[...], ws_ref[...],
                        preferred_element_type=jnp.float32) + bs_ref[...]
    o_ref[...] = _bn(jax.nn.relu(out), g_ref[...], be_ref[...])


def _h_update(lo3, hi3, wp3, h, ws, bs, g, be):
    blk = 512
    k = ACC_ROWS // blk
    acc_spec = pl.BlockSpec((1, blk, SW), lambda c, i: (c, i, 0))
    return pl.pallas_call(
        _hupd_body,
        grid=(NC, k),
        in_specs=[
            acc_spec, acc_spec, acc_spec,
            pl.BlockSpec((blk, HID), lambda c, i: (c * k + i, 0)),
            pl.BlockSpec((HID, HID), lambda c, i: (0, 0)),
            pl.BlockSpec((1, HID), lambda c, i: (0, 0)),
            pl.BlockSpec((1, HID), lambda c, i: (0, 0)),
            pl.BlockSpec((1, HID), lambda c, i: (0, 0)),
        ],
        out_specs=pl.BlockSpec((blk, HID), lambda c, i: (c * k + i, 0)),
        out_shape=jax.ShapeDtypeStruct((NP, HID), jnp.float32),
    )(lo3, hi3, wp3, h, ws, bs.reshape(1, HID), g.reshape(1, HID),
      be.reshape(1, HID))


def _head_body(h_ref, w1_ref, b1_ref, g1_ref, e1_ref, w2_ref, b2_ref,
               g2_ref, e2_ref, w3_ref, b3_ref, o_ref):
    z = jnp.dot(h_ref[...], w1_ref[...], preferred_element_type=jnp.float32)
    z = jax.nn.relu(_bn(z + b1_ref[...], g1_ref[...], e1_ref[...]))
    z = jnp.dot(z, w2_ref[...], preferred_element_type=jnp.float32)
    z = jax.nn.relu(_bn(z + b2_ref[...], g2_ref[...], e2_ref[...]))
    o_ref[...] = jnp.dot(z, w3_ref[...],
                         preferred_element_type=jnp.float32) + b3_ref[...]


def _head(h, p):
    blk = 512
    h2 = HID // 2
    return pl.pallas_call(
        _head_body,
        grid=(BS // blk,),
        in_specs=[
            pl.BlockSpec((blk, HID), lambda i: (i, 0)),
            pl.BlockSpec((HID, HID), lambda i: (0, 0)),
            pl.BlockSpec((1, HID), lambda i: (0, 0)),
            pl.BlockSpec((1, HID), lambda i: (0, 0)),
            pl.BlockSpec((1, HID), lambda i: (0, 0)),
            pl.BlockSpec((HID, h2), lambda i: (0, 0)),
            pl.BlockSpec((1, h2), lambda i: (0, 0)),
            pl.BlockSpec((1, h2), lambda i: (0, 0)),
            pl.BlockSpec((1, h2), lambda i: (0, 0)),
            pl.BlockSpec((h2, 1), lambda i: (0, 0)),
            pl.BlockSpec((1, 1), lambda i: (0, 0)),
        ],
        out_specs=pl.BlockSpec((blk, 1), lambda i: (i, 0)),
        out_shape=jax.ShapeDtypeStruct((BS, 1), jnp.float32),
    )(h, p['W1'], p['b1'].reshape(1, HID), p['g1'].reshape(1, HID),
      p['be1'].reshape(1, HID), p['W2'], p['b2'].reshape(1, h2),
      p['g2'].reshape(1, h2), p['be2'].reshape(1, h2),
      p['W3'], p['b3'].reshape(1, 1))


_MESH = plsc.VectorSubcoreMesh(core_axis_name="c", subcore_axis_name="s")
_SC_PARAMS = pltpu.CompilerParams(needs_layout_passes=False)


def _make_sc_gather(qw):
    @functools.partial(
        pl.kernel,
        out_type=[
            jax.ShapeDtypeStruct((EP, qw), jnp.float32),
            jax.ShapeDtypeStruct((EP, 2 * HID), jnp.float32),
        ],
        mesh=_MESH,
        scratch_types=[
            pltpu.VMEM((2, GB), jnp.int32),
            pltpu.VMEM((2, GB), jnp.int32),
            pltpu.VMEM((2, GB, qw), jnp.float32),
            pltpu.VMEM((2, GB, 2 * HID), jnp.float32),
            pltpu.SemaphoreType.DMA,
            pltpu.SemaphoreType.DMA,
            pltpu.SemaphoreType.DMA,
            pltpu.SemaphoreType.DMA,
            pltpu.SemaphoreType.DMA,
            pltpu.SemaphoreType.DMA,
        ],
        compiler_params=_SC_PARAMS,
    )
    def sc_gather(qt_hbm, kvt_hbm, dg_hbm, sg_hbm,
                  qd_out, kvs_out, di_v, si_v, qrows, kvrows,
                  isem0, isem1, gsem0, gsem1, wsem0, wsem1):
        wid = lax.axis_index("s") * NC + lax.axis_index("c")
        base = wid * PER_W
        ncheck = PER_W // GB
        njj = ncheck // 2
        isem = (isem0, isem1)
        gsem = (gsem0, gsem1)
        wsem = (wsem0, wsem1)

        def issue_idx(b, off):
            pltpu.async_copy(dg_hbm.at[pl.ds(off, GB)], di_v.at[b], isem[b])
            pltpu.async_copy(sg_hbm.at[pl.ds(off, GB)], si_v.at[b], isem[b])

        def wait_idx(b):
            pltpu.make_async_copy(dg_hbm.at[pl.ds(0, GB)], di_v.at[b],
                                  isem[b]).wait()
            pltpu.make_async_copy(sg_hbm.at[pl.ds(0, GB)], si_v.at[b],
                                  isem[b]).wait()

        def issue_gather(b):
            pltpu.async_copy(qt_hbm.at[di_v.at[b]], qrows.at[b], gsem[b])
            pltpu.async_copy(kvt_hbm.at[si_v.at[b]], kvrows.at[b], gsem[b])

        def wait_gather(b):
            pltpu.make_async_copy(qt_hbm.at[pl.ds(0, GB)], qrows.at[b],
                                  gsem[b]).wait()
            pltpu.make_async_copy(kvt_hbm.at[pl.ds(0, GB)], kvrows.at[b],
                                  gsem[b]).wait()

        def issue_wb(b, off):
            pltpu.async_copy(qrows.at[b], qd_out.at[pl.ds(off, GB)], wsem[b])
            pltpu.async_copy(kvrows.at[b], kvs_out.at[pl.ds(off, GB)], wsem[b])

        def wait_wb(b):
            pltpu.make_async_copy(qrows.at[b], qd_out.at[pl.ds(0, GB)],
                                  wsem[b]).wait()
            pltpu.make_async_copy(kvrows.at[b], kvs_out.at[pl.ds(0, GB)],
                                  wsem[b]).wait()

        issue_idx(0, base)

        def slot(jj, carry):
            for b in (0, 1):
                j2 = 2 * jj + b
                off = base + j2 * GB
                bp = 1 - b
                wait_idx(b)

                @pl.when(jj >= 1)
                def _():
                    wait_wb(b)

                issue_gather(b)
                if b == 1:
                    wait_gather(bp)
                    issue_wb(bp, off - GB)
                else:
                    @pl.when(jj >= 1)
                    def _():
                        wait_gather(bp)
                        issue_wb(bp, off - GB)
                if b == 0:
                    issue_idx(bp, off + GB)
                else:
                    @pl.when(jj < njj - 1)
                    def _():
                        issue_idx(bp, off + GB)
            return carry

        lax.fori_loop(0, njj, slot, 0)
        wait_gather(1)
        issue_wb(1, base + (ncheck - 1) * GB)
        wait_wb(0)
        wait_wb(1)

    return sc_gather


_sc_gather0 = _make_sc_gather(QW0)
_sc_gather1 = _make_sc_gather(QW1)


def _make_sc_scatter(narr):
    acc_t = jax.ShapeDtypeStruct((NC, ACC_ROWS, SW), jnp.float32)
    buf_t = pltpu.VMEM((2, SB, SW), jnp.float32)
    sh_t = pltpu.VMEM_SHARED((ACC_ROWS, SW), jnp.float32)

    @functools.partial(
        pl.kernel,
        out_type=[acc_t] * narr,
        mesh=_MESH,
        scratch_types=(
            [pltpu.VMEM((2, SB), jnp.int32), pltpu.VMEM((2, SB), jnp.int32)]
            + [buf_t] * narr + [sh_t] * narr
            + [pltpu.SemaphoreType.DMA, pltpu.SemaphoreType.DMA]
        ),
        compiler_params=_SC_PARAMS,
    )
    def sc_scatter(*refs):
        data_hbm = refs[:narr]
        ds_hbm = refs[narr]
        zrows_hbm = refs[narr + 1]
        outs = refs[narr + 2:2 * narr + 2]
        di_v = refs[2 * narr + 2]
        ai_v = refs[2 * narr + 3]
        bufs = refs[2 * narr + 4:3 * narr + 4]
        shs = refs[3 * narr + 4:4 * narr + 4]
        lsem = refs[4 * narr + 4:4 * narr + 6]
        cid = lax.axis_index("c")
        sid = lax.axis_index("s")
        nbase = cid * HALF
        ncheck = PER_T // SB
        njj = ncheck // 2

        zslice = pl.ds(sid * DRAIN, DRAIN)
        for sh in shs:
            pltpu.sync_copy(zrows_hbm, sh.at[zslice])
        plsc.subcore_barrier()

        def issue_loads(b, off):
            sl_rows = pl.ds(off, SB)
            pltpu.async_copy(ds_hbm.at[sl_rows], di_v.at[b], lsem[b])
            for src, buf in zip(data_hbm, bufs):
                pltpu.async_copy(src.at[sl_rows], buf.at[b], lsem[b])

        def wait_loads(b):
            sl0 = pl.ds(0, SB)
            pltpu.make_async_copy(ds_hbm.at[sl0], di_v.at[b], lsem[b]).wait()
            for src, buf in zip(data_hbm, bufs):
                pltpu.make_async_copy(src.at[sl0], buf.at[b], lsem[b]).wait()

        base_t = sid * PER_T
        issue_loads(0, base_t)

        def slot(jj, carry):
            for b in (0, 1):
                j2 = 2 * jj + b
                off = base_t + j2 * SB
                bp = 1 - b
                wait_loads(b)
                for k in range(SB // 16):
                    sl = pl.ds(k * 16, 16)
                    rel = di_v[b, sl] - nbase
                    ok = (rel >= 0) & (rel < HALF)
                    ai_v[b, sl] = jnp.where(ok, rel, HALF)
                if b == 0:
                    issue_loads(bp, off + SB)
                else:
                    @pl.when(jj < njj - 1)
                    def _():
                        issue_loads(bp, off + SB)
                for buf, sh in zip(bufs, shs):
                    pltpu.sync_copy(buf.at[b], sh.at[ai_v.at[b]], add=True)
            return carry

        lax.fori_loop(0, njj, slot, 0)
        plsc.subcore_barrier()
        for sh, out in zip(shs, outs):
            pltpu.sync_copy(sh.at[zslice], out.at[cid, zslice])

    return sc_scatter


_sc_scatter_mh = _make_sc_scatter(2)
_sc_scatter_w = _make_sc_scatter(1)


def kernel(x, edge_index, time, node_time, batch_size, params):
    src = edge_index[0]
    dst = edge_index[1]
    pad = EP - E
    pad_seg = ACC_ROWS - HALF
    dg = jnp.concatenate([dst, jnp.zeros((pad,), jnp.int32)])
    sg = jnp.concatenate([src, jnp.zeros((pad,), jnp.int32)])
    dgp = dg + pad_seg * (dg // HALF)
    sgp = sg + pad_seg * (sg // HALF)
    ds_ = jnp.concatenate([dst, jnp.full((pad,), jnp.int32(2 ** 20))])
    t_col = jnp.concatenate([time, jnp.zeros((pad,), jnp.float32)]).reshape(EP, 1)
    zseg = jnp.zeros((pad_seg,), jnp.float32)
    nt_col = jnp.concatenate(
        [node_time[:HALF], zseg, node_time[HALF:], zseg]).reshape(NP, 1)
    zrows = jnp.zeros((DRAIN, SW), jnp.float32)

    p = params
    h0 = _input_proj(x, p['W_in'], p['b_in'])
    zpad = jnp.zeros((pad_seg, HID), jnp.float32)
    h = jnp.concatenate([h0[:HALF], zpad, h0[HALF:], zpad], axis=0)
    rt_col = t_col
    for l in range(NL):
        qw = QW0 if l == 0 else QW1
        qt, kvt = _projections(h, nt_col, p['Wq'][l], p['bq'][l],
                               p['Wk'][l], p['bk'][l], p['Wv'][l], p['bv'][l],
                               qw)
        gather = _sc_gather0 if l == 0 else _sc_gather1
        qd, kvs = gather(qt, kvt, dgp, sgp)
        lo, hi, wp, rt_col = _edge_pass(qd, kvs, rt_col, t_col,
                                        p['basis_freq'], p['phase'],
                                        p['We'][l], qw)
        lo3, hi3 = _sc_scatter_mh(lo, hi, ds_, zrows)
        wp3, = _sc_scatter_w(wp, ds_, zrows)
        h = _h_update(lo3, hi3, wp3, h, p['Wskip'][l], p['bskip'][l],
                      p['gamma'][l], p['beta'][l])
    z = _head(h, p)
    return z[:, 0]


# speedup vs baseline: 2.3265x; 1.0177x over previous
import functools

import jax
import jax.numpy as jnp
from jax import lax
from jax.experimental import pallas as pl
from jax.experimental.pallas import tpu as pltpu
from jax.experimental.pallas import tpu_sc as plsc

N = 10000
E = 160000
IN = 128
HID = 256
HEADS = 4
C = HID // HEADS
TD = 64
NL = 2
BS = 4096

NC = 2
NS = 16
NW = NC * NS
EP = 163840
PER_W = EP // NW
GB = 64
PER_T = EP // NS
SB = 64
QW0 = 384
QW1 = 256
SW = 128
HALF = 5000
ACC_ROWS = 5120
DRAIN = ACC_ROWS // NS
NP = 2 * ACC_ROWS

_EPS_BN = 1e-5
_BN_SCALE = 1.0 / (1.0 + _EPS_BN) ** 0.5


def _bn(h, g, b):
    return h * (g * _BN_SCALE) + b


def _inproj_body(x_ref, w_ref, b_ref, o_ref):
    o_ref[...] = jax.nn.relu(
        jnp.dot(x_ref[...], w_ref[...], preferred_element_type=jnp.float32)
        + b_ref[...])


def _input_proj(x, w, b):
    blk = 2000
    return pl.pallas_call(
        _inproj_body,
        grid=(N // blk,),
        in_specs=[
            pl.BlockSpec((blk, IN), lambda i: (i, 0)),
            pl.BlockSpec((IN, HID), lambda i: (0, 0)),
            pl.BlockSpec((1, HID), lambda i: (0, 0)),
        ],
        out_specs=pl.BlockSpec((blk, HID), lambda i: (i, 0)),
        out_shape=jax.ShapeDtypeStruct((N, HID), jnp.float32),
    )(x, w, b.reshape(1, HID))


def _make_proj_body(qw):
    def body(h_ref, nt_ref, wq_ref, bq_ref, wk_ref, bk_ref, wv_ref, bv_ref,
             qt_ref, kvt_ref):
        h = h_ref[...]
        q = jnp.dot(h, wq_ref[...], preferred_element_type=jnp.float32) + bq_ref[...]
        k = jnp.dot(h, wk_ref[...], preferred_element_type=jnp.float32) + bk_ref[...]
        v = jnp.dot(h, wv_ref[...], preferred_element_type=jnp.float32) + bv_ref[...]
        if qw > HID:
            pad = jnp.zeros((h.shape[0], qw - HID - 1), jnp.float32)
            qt_ref[...] = jnp.concatenate([q, nt_ref[...], pad], axis=1)
        else:
            qt_ref[...] = q
        kvt_ref[...] = jnp.concatenate([k, v], axis=1)
    return body


def _projections(h, nt, wq, bq, wk, bk, wv, bv, qw):
    blk = 1024
    w_spec = pl.BlockSpec((HID, HID), lambda i: (0, 0))
    b_spec = pl.BlockSpec((1, HID), lambda i: (0, 0))
    return pl.pallas_call(
        _make_proj_body(qw),
        grid=(NP // blk,),
        in_specs=[
            pl.BlockSpec((blk, HID), lambda i: (i, 0)),
            pl.BlockSpec((blk, 1), lambda i: (i, 0)),
            w_spec, b_spec, w_spec, b_spec, w_spec, b_spec,
        ],
        out_specs=[
            pl.BlockSpec((blk, qw), lambda i: (i, 0)),
            pl.BlockSpec((blk, 2 * HID), lambda i: (i, 0)),
        ],
        out_shape=[
            jax.ShapeDtypeStruct((NP, qw), jnp.float32),
            jax.ShapeDtypeStruct((NP, 2 * HID), jnp.float32),
        ],
    )(h, nt, wq, bq.reshape(1, HID), wk, bk.reshape(1, HID),
      wv, bv.reshape(1, HID))


_BE = 2048


def _enc(rel_t, bf_ref, ph_ref):
    return jnp.cos(rel_t * bf_ref[...] + ph_ref[...])


_ASHIFT = 30.0


def _make_edge_body(qw):
    def body(qd_ref, kv_ref, rt_ref, t_ref, bf_ref, ph_ref, we_ref,
             lo_ref, hi_ref, wp_ref, rt_out_ref):
        qd = qd_ref[...]
        if qw > HID:
            rel_t = qd[:, HID:HID + 1] - t_ref[...]
        else:
            rel_t = rt_ref[...]
        em = jnp.dot(_enc(rel_t, bf_ref, ph_ref), we_ref[...],
                     preferred_element_type=jnp.float32)
        kv = kv_ref[...]
        kk = kv[:, :HID] + em
        prod = (qd[:, :HID] * kk).reshape(_BE, HEADS, C)
        scale = 1.0 / (C ** 0.5)
        alpha = prod.sum(axis=-1) * scale
        w = jnp.exp(alpha - _ASHIFT)
        wb = jnp.broadcast_to(w.reshape(_BE, HEADS, 1), (_BE, HEADS, C))
        msg = (kv[:, HID:] + em) * wb.reshape(_BE, HID)
        lo_ref[...] = msg[:, :SW]
        hi_ref[...] = msg[:, SW:]
        wp_ref[...] = jnp.concatenate(
            [w, jnp.zeros((_BE, SW - HEADS), jnp.float32)], axis=1)
        rt_out_ref[...] = rel_t
    return body


def _edge_pass(qd, kvs, rt_col, t_col, bf, ph, we, qw):
    grid = EP // _BE
    return pl.pallas_call(
        _make_edge_body(qw),
        grid=(grid,),
        in_specs=[
            pl.BlockSpec((_BE, qw), lambda i: (i, 0)),
            pl.BlockSpec((_BE, 2 * HID), lambda i: (i, 0)),
            pl.BlockSpec((_BE, 1), lambda i: (i, 0)),
            pl.BlockSpec((_BE, 1), lambda i: (i, 0)),
            pl.BlockSpec((1, TD), lambda i: (0, 0)),
            pl.BlockSpec((1, TD), lambda i: (0, 0)),
            pl.BlockSpec((TD, HID), lambda i: (0, 0)),
        ],
        out_specs=[
            pl.BlockSpec((_BE, SW), lambda i: (i, 0)),
            pl.BlockSpec((_BE, SW), lambda i: (i, 0)),
            pl.BlockSpec((_BE, SW), lambda i: (i, 0)),
            pl.BlockSpec((_BE, 1), lambda i: (i, 0)),
        ],
        out_shape=[
            jax.ShapeDtypeStruct((EP, SW), jnp.float32),
            jax.ShapeDtypeStruct((EP, SW), jnp.float32),
            jax.ShapeDtypeStruct((EP, SW), jnp.float32),
            jax.ShapeDtypeStruct((EP, 1), jnp.float32),
        ],
    )(qd, kvs, rt_col, t_col, bf.reshape(1, TD), ph.reshape(1, TD), we)


def _hupd_body(lo_ref, hi_ref, wp_ref, h_ref, ws_ref, bs_ref, g_ref, be_ref,
               o_ref):
    lo = lo_ref[0]
    hi = hi_ref[0]
    blk = lo.shape[0]
    msg = jnp.concatenate([lo, hi], axis=1)
    den = wp_ref[0][:, :HEADS]
    den_b = jnp.broadcast_to(den.reshape(blk, HEADS, 1), (blk, HEADS, C))
    den_b = den_b.reshape(blk, HID)
    out = msg / jnp.maximum(den_b, 1e-30)
    out = out + jnp.dot(h_ref[...], ws_ref[...],
                        preferred_element_type=jnp.float32) + bs_ref[...]
    o_ref[...] = _bn(jax.nn.relu(out), g_ref[...], be_ref[...])


def _h_update(lo3, hi3, wp3, h, ws, bs, g, be):
    blk = 512
    k = ACC_ROWS // blk
    acc_spec = pl.BlockSpec((1, blk, SW), lambda c, i: (c, i, 0))
    return pl.pallas_call(
        _hupd_body,
        grid=(NC, k),
        in_specs=[
            acc_spec, acc_spec, acc_spec,
            pl.BlockSpec((blk, HID), lambda c, i: (c * k + i, 0)),
            pl.BlockSpec((HID, HID), lambda c, i: (0, 0)),
            pl.BlockSpec((1, HID), lambda c, i: (0, 0)),
            pl.BlockSpec((1, HID), lambda c, i: (0, 0)),
            pl.BlockSpec((1, HID), lambda c, i: (0, 0)),
        ],
        out_specs=pl.BlockSpec((blk, HID), lambda c, i: (c * k + i, 0)),
        out_shape=jax.ShapeDtypeStruct((NP, HID), jnp.float32),
    )(lo3, hi3, wp3, h, ws, bs.reshape(1, HID), g.reshape(1, HID),
      be.reshape(1, HID))


def _head_body(h_ref, w1_ref, b1_ref, g1_ref, e1_ref, w2_ref, b2_ref,
               g2_ref, e2_ref, w3_ref, b3_ref, o_ref):
    z = jnp.dot(h_ref[...], w1_ref[...], preferred_element_type=jnp.float32)
    z = jax.nn.relu(_bn(z + b1_ref[...], g1_ref[...], e1_ref[...]))
    z = jnp.dot(z, w2_ref[...], preferred_element_type=jnp.float32)
    z = jax.nn.relu(_bn(z + b2_ref[...], g2_ref[...], e2_ref[...]))
    o_ref[...] = jnp.dot(z, w3_ref[...],
                         preferred_element_type=jnp.float32) + b3_ref[...]


def _head(h, p):
    blk = 512
    h2 = HID // 2
    return pl.pallas_call(
        _head_body,
        grid=(BS // blk,),
        in_specs=[
            pl.BlockSpec((blk, HID), lambda i: (i, 0)),
            pl.BlockSpec((HID, HID), lambda i: (0, 0)),
            pl.BlockSpec((1, HID), lambda i: (0, 0)),
            pl.BlockSpec((1, HID), lambda i: (0, 0)),
            pl.BlockSpec((1, HID), lambda i: (0, 0)),
            pl.BlockSpec((HID, h2), lambda i: (0, 0)),
            pl.BlockSpec((1, h2), lambda i: (0, 0)),
            pl.BlockSpec((1, h2), lambda i: (0, 0)),
            pl.BlockSpec((1, h2), lambda i: (0, 0)),
            pl.BlockSpec((h2, 1), lambda i: (0, 0)),
            pl.BlockSpec((1, 1), lambda i: (0, 0)),
        ],
        out_specs=pl.BlockSpec((blk, 1), lambda i: (i, 0)),
        out_shape=jax.ShapeDtypeStruct((BS, 1), jnp.float32),
    )(h, p['W1'], p['b1'].reshape(1, HID), p['g1'].reshape(1, HID),
      p['be1'].reshape(1, HID), p['W2'], p['b2'].reshape(1, h2),
      p['g2'].reshape(1, h2), p['be2'].reshape(1, h2),
      p['W3'], p['b3'].reshape(1, 1))


_MESH = plsc.VectorSubcoreMesh(core_axis_name="c", subcore_axis_name="s")
_SC_PARAMS = pltpu.CompilerParams(needs_layout_passes=False)


def _make_sc_gather(qw):
    @functools.partial(
        pl.kernel,
        out_type=[
            jax.ShapeDtypeStruct((EP, qw), jnp.float32),
            jax.ShapeDtypeStruct((EP, 2 * HID), jnp.float32),
        ],
        mesh=_MESH,
        scratch_types=[
            pltpu.VMEM((2, GB), jnp.int32),
            pltpu.VMEM((2, GB), jnp.int32),
            pltpu.VMEM((2, GB, qw), jnp.float32),
            pltpu.VMEM((2, GB, 2 * HID), jnp.float32),
            pltpu.SemaphoreType.DMA,
            pltpu.SemaphoreType.DMA,
            pltpu.SemaphoreType.DMA,
            pltpu.SemaphoreType.DMA,
            pltpu.SemaphoreType.DMA,
            pltpu.SemaphoreType.DMA,
        ],
        compiler_params=_SC_PARAMS,
    )
    def sc_gather(qt_hbm, kvt_hbm, dg_hbm, sg_hbm,
                  qd_out, kvs_out, di_v, si_v, qrows, kvrows,
                  isem0, isem1, gsem0, gsem1, wsem0, wsem1):
        wid = lax.axis_index("s") * NC + lax.axis_index("c")
        base = wid * PER_W
        ncheck = PER_W // GB
        njj = ncheck // 2
        isem = (isem0, isem1)
        gsem = (gsem0, gsem1)
        wsem = (wsem0, wsem1)

        def issue_idx(b, off):
            pltpu.async_copy(dg_hbm.at[pl.ds(off, GB)], di_v.at[b], isem[b])
            pltpu.async_copy(sg_hbm.at[pl.ds(off, GB)], si_v.at[b], isem[b])

        def wait_idx(b):
            pltpu.make_async_copy(dg_hbm.at[pl.ds(0, GB)], di_v.at[b],
                                  isem[b]).wait()
            pltpu.make_async_copy(sg_hbm.at[pl.ds(0, GB)], si_v.at[b],
                                  isem[b]).wait()

        def issue_gather(b):
            pltpu.async_copy(qt_hbm.at[di_v.at[b]], qrows.at[b], gsem[b])
            pltpu.async_copy(kvt_hbm.at[si_v.at[b]], kvrows.at[b], gsem[b])

        def wait_gather(b):
            pltpu.make_async_copy(qt_hbm.at[pl.ds(0, GB)], qrows.at[b],
                                  gsem[b]).wait()
            pltpu.make_async_copy(kvt_hbm.at[pl.ds(0, GB)], kvrows.at[b],
                                  gsem[b]).wait()

        def issue_wb(b, off):
            pltpu.async_copy(qrows.at[b], qd_out.at[pl.ds(off, GB)], wsem[b])
            pltpu.async_copy(kvrows.at[b], kvs_out.at[pl.ds(off, GB)], wsem[b])

        def wait_wb(b):
            pltpu.make_async_copy(qrows.at[b], qd_out.at[pl.ds(0, GB)],
                                  wsem[b]).wait()
            pltpu.make_async_copy(kvrows.at[b], kvs_out.at[pl.ds(0, GB)],
                                  wsem[b]).wait()

        issue_idx(0, base)

        def slot(jj, carry):
            for b in (0, 1):
                j2 = 2 * jj + b
                off = base + j2 * GB
                bp = 1 - b
                wait_idx(b)

                @pl.when(jj >= 1)
                def _():
                    wait_wb(b)

                issue_gather(b)
                if b == 1:
                    wait_gather(bp)
                    issue_wb(bp, off - GB)
                else:
                    @pl.when(jj >= 1)
                    def _():
                        wait_gather(bp)
                        issue_wb(bp, off - GB)
                if b == 0:
                    issue_idx(bp, off + GB)
                else:
                    @pl.when(jj < njj - 1)
                    def _():
                        issue_idx(bp, off + GB)
            return carry

        lax.fori_loop(0, njj, slot, 0)
        wait_gather(1)
        issue_wb(1, base + (ncheck - 1) * GB)
        wait_wb(0)
        wait_wb(1)

    return sc_gather


_sc_gather0 = _make_sc_gather(QW0)
_sc_gather1 = _make_sc_gather(QW1)


def _make_sc_scatter(narr, SB):
    acc_t = jax.ShapeDtypeStruct((NC, ACC_ROWS, SW), jnp.float32)
    buf_t = pltpu.VMEM((2, SB, SW), jnp.float32)
    sh_t = pltpu.VMEM_SHARED((ACC_ROWS, SW), jnp.float32)

    @functools.partial(
        pl.kernel,
        out_type=[acc_t] * narr,
        mesh=_MESH,
        scratch_types=(
            [pltpu.VMEM((2, SB), jnp.int32), pltpu.VMEM((2, SB), jnp.int32)]
            + [buf_t] * narr + [sh_t] * narr
            + [pltpu.SemaphoreType.DMA, pltpu.SemaphoreType.DMA]
        ),
        compiler_params=_SC_PARAMS,
    )
    def sc_scatter(*refs):
        data_hbm = refs[:narr]
        ds_hbm = refs[narr]
        zrows_hbm = refs[narr + 1]
        outs = refs[narr + 2:2 * narr + 2]
        di_v = refs[2 * narr + 2]
        ai_v = refs[2 * narr + 3]
        bufs = refs[2 * narr + 4:3 * narr + 4]
        shs = refs[3 * narr + 4:4 * narr + 4]
        lsem = refs[4 * narr + 4:4 * narr + 6]
        cid = lax.axis_index("c")
        sid = lax.axis_index("s")
        nbase = cid * HALF
        ncheck = PER_T // SB
        njj = ncheck // 2

        zslice = pl.ds(sid * DRAIN, DRAIN)
        for sh in shs:
            pltpu.sync_copy(zrows_hbm, sh.at[zslice])
        plsc.subcore_barrier()

        def issue_loads(b, off):
            sl_rows = pl.ds(off, SB)
            pltpu.async_copy(ds_hbm.at[sl_rows], di_v.at[b], lsem[b])
            for src, buf in zip(data_hbm, bufs):
                pltpu.async_copy(src.at[sl_rows], buf.at[b], lsem[b])

        def wait_loads(b):
            sl0 = pl.ds(0, SB)
            pltpu.make_async_copy(ds_hbm.at[sl0], di_v.at[b], lsem[b]).wait()
            for src, buf in zip(data_hbm, bufs):
                pltpu.make_async_copy(src.at[sl0], buf.at[b], lsem[b]).wait()

        base_t = sid * PER_T
        issue_loads(0, base_t)

        def slot(jj, carry):
            for b in (0, 1):
                j2 = 2 * jj + b
                off = base_t + j2 * SB
                bp = 1 - b
                wait_loads(b)
                for k in range(SB // 16):
                    sl = pl.ds(k * 16, 16)
                    rel = di_v[b, sl] - nbase
                    ok = (rel >= 0) & (rel < HALF)
                    ai_v[b, sl] = jnp.where(ok, rel, HALF)
                if b == 0:
                    issue_loads(bp, off + SB)
                else:
                    @pl.when(jj < njj - 1)
                    def _():
                        issue_loads(bp, off + SB)
                for buf, sh in zip(bufs, shs):
                    pltpu.sync_copy(buf.at[b], sh.at[ai_v.at[b]], add=True)
            return carry

        lax.fori_loop(0, njj, slot, 0)
        plsc.subcore_barrier()
        for sh, out in zip(shs, outs):
            pltpu.sync_copy(sh.at[zslice], out.at[cid, zslice])

    return sc_scatter


_sc_scatter_mh = _make_sc_scatter(2, 80)
_sc_scatter_w = _make_sc_scatter(1, SB * 2)


def kernel(x, edge_index, time, node_time, batch_size, params):
    src = edge_index[0]
    dst = edge_index[1]
    pad = EP - E
    pad_seg = ACC_ROWS - HALF
    dg = jnp.concatenate([dst, jnp.zeros((pad,), jnp.int32)])
    sg = jnp.concatenate([src, jnp.zeros((pad,), jnp.int32)])
    dgp = dg + pad_seg * (dg // HALF)
    sgp = sg + pad_seg * (sg // HALF)
    ds_ = jnp.concatenate([dst, jnp.full((pad,), jnp.int32(2 ** 20))])
    t_col = jnp.concatenate([time, jnp.zeros((pad,), jnp.float32)]).reshape(EP, 1)
    zseg = jnp.zeros((pad_seg,), jnp.float32)
    nt_col = jnp.concatenate(
        [node_time[:HALF], zseg, node_time[HALF:], zseg]).reshape(NP, 1)
    zrows = jnp.zeros((DRAIN, SW), jnp.float32)

    p = params
    h0 = _input_proj(x, p['W_in'], p['b_in'])
    zpad = jnp.zeros((pad_seg, HID), jnp.float32)
    h = jnp.concatenate([h0[:HALF], zpad, h0[HALF:], zpad], axis=0)
    rt_col = t_col
    for l in range(NL):
        qw = QW0 if l == 0 else QW1
        qt, kvt = _projections(h, nt_col, p['Wq'][l], p['bq'][l],
                               p['Wk'][l], p['bk'][l], p['Wv'][l], p['bv'][l],
                               qw)
        gather = _sc_gather0 if l == 0 else _sc_gather1
        qd, kvs = gather(qt, kvt, dgp, sgp)
        lo, hi, wp, rt_col = _edge_pass(qd, kvs, rt_col, t_col,
                                        p['basis_freq'], p['phase'],
                                        p['We'][l], qw)
        lo3, hi3 = _sc_scatter_mh(lo, hi, ds_, zrows)
        wp3, = _sc_scatter_w(wp, ds_, zrows)
        h = _h_update(lo3, hi3, wp3, h, p['Wskip'][l], p['bskip'][l],
                      p['gamma'][l], p['beta'][l])
    z = _head(h, p)
    return z[:, 0]
